# Initial kernel scaffold; baseline (speedup 1.0000x reference)
import jax
import jax.numpy as jnp
from jax.experimental import pallas as pl


def kernel(h, coord, edge_index, coord_diff, edge_attr, W1, b1, W2, b2, W3):
    raise NotImplementedError("write your pallas kernel here")


# speedup vs baseline: 2.5103x; 2.5103x over previous
import functools

import jax
import jax.numpy as jnp
from jax import lax
from jax.experimental import pallas as pl
from jax.experimental.pallas import tpu as pltpu
from jax.experimental.pallas import tpu_sc as plsc

NC = 2
NS = 16
NW = NC * NS


def _precompute_tc(n, hnf, bn):
    def body(h_ref, wa_ref, wb_ref, a_ref, b_ref):
        hb = h_ref[...]
        a_ref[...] = jnp.dot(hb, wa_ref[...], preferred_element_type=jnp.float32)
        b_ref[...] = jnp.dot(hb, wb_ref[...], preferred_element_type=jnp.float32)

    return pl.pallas_call(
        body,
        grid=(n // bn,),
        in_specs=[
            pl.BlockSpec((bn, hnf), lambda i: (i, 0)),
            pl.BlockSpec((hnf, hnf), lambda i: (0, 0)),
            pl.BlockSpec((hnf, hnf), lambda i: (0, 0)),
        ],
        out_specs=[
            pl.BlockSpec((bn, hnf), lambda i: (i, 0)),
            pl.BlockSpec((bn, hnf), lambda i: (i, 0)),
        ],
        out_shape=[
            jax.ShapeDtypeStruct((n, hnf), jnp.float32),
            jax.ShapeDtypeStruct((n, hnf), jnp.float32),
        ],
    )


def _gather_add_sc(e, hnf, k):
    ew = e // NW
    nch = ew // k
    mesh = plsc.VectorSubcoreMesh(core_axis_name="c", subcore_axis_name="s")

    @functools.partial(
        pl.kernel,
        out_type=jax.ShapeDtypeStruct((e, hnf), jnp.float32),
        mesh=mesh,
        scratch_types=[
            pltpu.VMEM((k,), jnp.int32),
            pltpu.VMEM((k,), jnp.int32),
            pltpu.VMEM((k, hnf), jnp.float32),
            pltpu.VMEM((k, hnf), jnp.float32),
            pltpu.SemaphoreType.DMA,
            pltpu.SemaphoreType.DMA,
        ],
    )
    def gather_add(a_hbm, b_hbm, row_hbm, col_hbm, s_hbm,
                   idr, idc, bufa, bufb, sema, semb):
        cid = lax.axis_index("c")
        sid = lax.axis_index("s")
        wid = sid * NC + cid
        base = wid * ew

        def chunk(j, carry):
            off = base + j * k
            pltpu.sync_copy(row_hbm.at[pl.ds(off, k)], idr)
            pltpu.sync_copy(col_hbm.at[pl.ds(off, k)], idc)
            ca = pltpu.async_copy(a_hbm.at[idr], bufa, sema)
            cb = pltpu.async_copy(b_hbm.at[idc], bufb, semb)
            ca.wait()
            cb.wait()

            def addrow(r, c2):
                for t in range(hnf // 16):
                    sl = pl.ds(t * 16, 16)
                    bufa[r, sl] = bufa[r, sl] + bufb[r, sl]
                return c2

            lax.fori_loop(0, k, addrow, 0)
            pltpu.sync_copy(bufa, s_hbm.at[pl.ds(off, k)])
            return carry

        lax.fori_loop(0, nch, chunk, 0)

    return gather_add


def _mlp_tc(e, hnf, be):
    def body(s_ref, ea_ref, cd_ref, w1c_ref, b1_ref, w2_ref, b2_ref, w3_ref,
             o_ref):
        u = (s_ref[...]
             + jnp.dot(ea_ref[...], w1c_ref[...],
                       preferred_element_type=jnp.float32)
             + b1_ref[...])
        u = u * lax.logistic(u)
        x = jnp.dot(u, w2_ref[...], preferred_element_type=jnp.float32) + b2_ref[...]
        x = x * lax.logistic(x)
        m = jnp.sum(x * w3_ref[...], axis=1, keepdims=True)
        o_ref[...] = cd_ref[...] * m

    return pl.pallas_call(
        body,
        grid=(e // be,),
        in_specs=[
            pl.BlockSpec((be, hnf), lambda i: (i, 0)),
            pl.BlockSpec((be, 8), lambda i: (i, 0)),
            pl.BlockSpec((be, 4), lambda i: (i, 0)),
            pl.BlockSpec((8, hnf), lambda i: (0, 0)),
            pl.BlockSpec((1, hnf), lambda i: (0, 0)),
            pl.BlockSpec((hnf, hnf), lambda i: (0, 0)),
            pl.BlockSpec((1, hnf), lambda i: (0, 0)),
            pl.BlockSpec((1, hnf), lambda i: (0, 0)),
        ],
        out_specs=pl.BlockSpec((be, 4), lambda i: (i, 0)),
        out_shape=jax.ShapeDtypeStruct((e, 4), jnp.float32),
    )


def _scatter_sc(e, n, k):
    ew = e // NW
    nch = ew // k
    mesh = plsc.VectorSubcoreMesh(core_axis_name="c", subcore_axis_name="s")

    @functools.partial(
        pl.kernel,
        out_type=jax.ShapeDtypeStruct((NC, n, 4), jnp.float32),
        mesh=mesh,
        scratch_types=[
            pltpu.VMEM((k,), jnp.int32),
            pltpu.VMEM((k, 4), jnp.float32),
            pltpu.VMEM_SHARED((n, 4), jnp.float32),
        ],
    )
    def scatter(trans_hbm, row_hbm, zero_hbm, out_hbm, idx, tbuf, acc):
        cid = lax.axis_index("c")
        sid = lax.axis_index("s")
        wid = sid * NC + cid
        base = wid * ew

        @pl.when(sid == 0)
        def _():
            pltpu.sync_copy(zero_hbm, acc)

        plsc.subcore_barrier()

        def chunk(j, carry):
            off = base + j * k
            pltpu.sync_copy(row_hbm.at[pl.ds(off, k)], idx)
            pltpu.sync_copy(trans_hbm.at[pl.ds(off, k)], tbuf)
            pltpu.sync_copy(tbuf, acc.at[idx], add=True)
            return carry

        lax.fori_loop(0, nch, chunk, 0)
        plsc.subcore_barrier()

        @pl.when(sid == 0)
        def _():
            pltpu.sync_copy(acc, out_hbm.at[cid])

    return scatter


def _combine_tc(n):
    def body(p_ref, c_ref, o_ref):
        p = p_ref[0] + p_ref[1]
        o_ref[...] = c_ref[...] + p[:, :3] / jnp.float32(100.0)

    return pl.pallas_call(
        body,
        out_shape=jax.ShapeDtypeStruct((n, 3), jnp.float32),
    )


def kernel(h, coord, edge_index, coord_diff, edge_attr, W1, b1, W2, b2, W3):
    n, hnf = h.shape
    e = edge_index.shape[1]
    f32 = jnp.float32
    row = edge_index[0].astype(jnp.int32)
    col = edge_index[1].astype(jnp.int32)
    w1t = W1.T.astype(f32)
    w1a = w1t[:hnf]
    w1b = w1t[hnf:2 * hnf]
    nea = w1t.shape[0] - 2 * hnf
    w1c = jnp.zeros((8, hnf), f32).at[:nea].set(w1t[2 * hnf:])
    ea8 = jnp.zeros((e, 8), f32).at[:, :nea].set(edge_attr.astype(f32))
    cd4 = jnp.zeros((e, 4), f32).at[:, :3].set(coord_diff.astype(f32))
    b1r = b1.reshape(1, hnf).astype(f32)
    b2r = b2.reshape(1, hnf).astype(f32)
    w2t = W2.T.astype(f32)
    w3r = W3.reshape(1, hnf).astype(f32)
    zeros_n4 = jnp.zeros((n, 4), f32)

    A, B = _precompute_tc(n, hnf, 2000)(h.astype(f32), w1a, w1b)
    S = _gather_add_sc(e, hnf, 80)(A, B, row, col)
    trans = _mlp_tc(e, hnf, 1280)(S, ea8, cd4, w1c, b1r, w2t, b2r, w3r)
    parts = _scatter_sc(e, n, 80)(trans, row, zeros_n4)
    return _combine_tc(n)(parts, coord.astype(f32))


# speedup vs baseline: 2.5339x; 1.0094x over previous
import functools

import jax
import jax.numpy as jnp
from jax import lax
from jax.experimental import pallas as pl
from jax.experimental.pallas import tpu as pltpu
from jax.experimental.pallas import tpu_sc as plsc

NC = 2
NS = 16
NW = NC * NS


def _precompute_tc(n, hnf, bn):
    def body(h_ref, wa_ref, wb_ref, a_ref, b_ref):
        hb = h_ref[...]
        a_ref[...] = jnp.dot(hb, wa_ref[...], preferred_element_type=jnp.float32)
        b_ref[...] = jnp.dot(hb, wb_ref[...], preferred_element_type=jnp.float32)

    return pl.pallas_call(
        body,
        grid=(n // bn,),
        in_specs=[
            pl.BlockSpec((bn, hnf), lambda i: (i, 0)),
            pl.BlockSpec((hnf, hnf), lambda i: (0, 0)),
            pl.BlockSpec((hnf, hnf), lambda i: (0, 0)),
        ],
        out_specs=[
            pl.BlockSpec((bn, hnf), lambda i: (i, 0)),
            pl.BlockSpec((bn, hnf), lambda i: (i, 0)),
        ],
        out_shape=[
            jax.ShapeDtypeStruct((n, hnf), jnp.float32),
            jax.ShapeDtypeStruct((n, hnf), jnp.float32),
        ],
    )


def _gather_add_sc(e, hnf, k):
    ew = e // NW
    nch = ew // k
    mesh = plsc.VectorSubcoreMesh(core_axis_name="c", subcore_axis_name="s")

    @functools.partial(
        pl.kernel,
        out_type=jax.ShapeDtypeStruct((e, hnf), jnp.float32),
        mesh=mesh,
        scratch_types=[
            pltpu.VMEM((ew,), jnp.int32),
            pltpu.VMEM((ew,), jnp.int32),
            pltpu.VMEM((k, hnf), jnp.float32),
            pltpu.VMEM((k, hnf), jnp.float32),
            pltpu.VMEM((k, hnf), jnp.float32),
            pltpu.VMEM((k, hnf), jnp.float32),
            pltpu.VMEM((k, hnf), jnp.float32),
            pltpu.VMEM((k, hnf), jnp.float32),
            pltpu.SemaphoreType.DMA,
            pltpu.SemaphoreType.DMA,
            pltpu.SemaphoreType.DMA,
            pltpu.SemaphoreType.DMA,
            pltpu.SemaphoreType.DMA,
            pltpu.SemaphoreType.DMA,
        ],
    )
    def gather_add(a_hbm, b_hbm, row_hbm, col_hbm, s_hbm,
                   idr, idc, ba0, ba1, bb0, bb1, sb0, sb1,
                   sma0, sma1, smb0, smb1, smw0, smw1):
        cid = lax.axis_index("c")
        sid = lax.axis_index("s")
        wid = sid * NC + cid
        base = wid * ew
        pltpu.sync_copy(row_hbm.at[pl.ds(base, ew)], idr)
        pltpu.sync_copy(col_hbm.at[pl.ds(base, ew)], idc)
        ba = (ba0, ba1)
        bb = (bb0, bb1)
        sb = (sb0, sb1)
        sma = (sma0, sma1)
        smb = (smb0, smb1)
        smw = (smw0, smw1)

        def issue(g, b):
            pltpu.make_async_copy(
                a_hbm.at[idr.at[pl.ds(g * k, k)]], ba[b], sma[b]).start()
            pltpu.make_async_copy(
                b_hbm.at[idc.at[pl.ds(g * k, k)]], bb[b], smb[b]).start()

        def addrows(b):
            def addrow(r, c2):
                for tt in range(hnf // 16):
                    sl = pl.ds(tt * 16, 16)
                    sb[b][r, sl] = ba[b][r, sl] + bb[b][r, sl]
                return c2

            lax.fori_loop(0, k, addrow, 0)

        def step(g, b):
            @pl.when(g + 1 < nch)
            def _():
                issue(g + 1, 1 - b)

            pltpu.make_async_copy(
                a_hbm.at[idr.at[pl.ds(g * k, k)]], ba[b], sma[b]).wait()
            pltpu.make_async_copy(
                b_hbm.at[idc.at[pl.ds(g * k, k)]], bb[b], smb[b]).wait()

            @pl.when(g >= 2)
            def _():
                pltpu.make_async_copy(
                    sb[b], s_hbm.at[pl.ds(base, k)], smw[b]).wait()

            addrows(b)
            pltpu.make_async_copy(
                sb[b], s_hbm.at[pl.ds(base + g * k, k)], smw[b]).start()

        issue(0, 0)

        def pair(t, carry):
            for b in range(2):
                step(t * 2 + b, b)
            return carry

        lax.fori_loop(0, nch // 2, pair, 0)
        if nch % 2:
            step(nch - 1, 0)
        pltpu.make_async_copy(sb0, s_hbm.at[pl.ds(base, k)], smw0).wait()
        pltpu.make_async_copy(sb1, s_hbm.at[pl.ds(base, k)], smw1).wait()

    return gather_add


def _mlp_tc(e, hnf, be):
    def body(s_ref, ea_ref, cd_ref, w1c_ref, b1_ref, w2_ref, b2_ref, w3_ref,
             o_ref):
        u = (s_ref[...]
             + jnp.dot(ea_ref[...], w1c_ref[...],
                       preferred_element_type=jnp.float32)
             + b1_ref[...])
        u = u * lax.logistic(u)
        x = jnp.dot(u, w2_ref[...], preferred_element_type=jnp.float32) + b2_ref[...]
        x = x * lax.logistic(x)
        m = jnp.sum(x * w3_ref[...], axis=1, keepdims=True)
        o_ref[...] = cd_ref[...] * m

    return pl.pallas_call(
        body,
        grid=(e // be,),
        in_specs=[
            pl.BlockSpec((be, hnf), lambda i: (i, 0)),
            pl.BlockSpec((be, 8), lambda i: (i, 0)),
            pl.BlockSpec((be, 4), lambda i: (i, 0)),
            pl.BlockSpec((8, hnf), lambda i: (0, 0)),
            pl.BlockSpec((1, hnf), lambda i: (0, 0)),
            pl.BlockSpec((hnf, hnf), lambda i: (0, 0)),
            pl.BlockSpec((1, hnf), lambda i: (0, 0)),
            pl.BlockSpec((1, hnf), lambda i: (0, 0)),
        ],
        out_specs=pl.BlockSpec((be, 4), lambda i: (i, 0)),
        out_shape=jax.ShapeDtypeStruct((e, 4), jnp.float32),
    )


def _scatter_sc(e, n, k):
    ew = e // NW
    nch = ew // k
    mesh = plsc.VectorSubcoreMesh(core_axis_name="c", subcore_axis_name="s")

    @functools.partial(
        pl.kernel,
        out_type=jax.ShapeDtypeStruct((NC, n, 4), jnp.float32),
        mesh=mesh,
        compiler_params=pltpu.CompilerParams(use_tc_tiling_on_sc=False),
        scratch_types=[
            pltpu.VMEM((nch, k), jnp.int32),
            pltpu.VMEM((ew, 4), jnp.float32),
            pltpu.VMEM_SHARED((n, 4), jnp.float32),
        ],
    )
    def scatter(trans_hbm, row2d_hbm, zero_hbm, out_hbm, idx2, tbuf, acc):
        cid = lax.axis_index("c")
        sid = lax.axis_index("s")
        wid = sid * NC + cid

        @pl.when(sid == 0)
        def _():
            pltpu.sync_copy(zero_hbm, acc)

        pltpu.sync_copy(row2d_hbm.at[pl.ds(wid * nch, nch)], idx2)
        pltpu.sync_copy(trans_hbm.at[pl.ds(wid * ew, ew)], tbuf)
        plsc.subcore_barrier()

        def chunk(j, carry):
            pltpu.sync_copy(tbuf.at[pl.ds(j * k, k)], acc.at[idx2.at[j]],
                            add=True)
            return carry

        lax.fori_loop(0, nch, chunk, 0)
        plsc.subcore_barrier()

        @pl.when(sid == 0)
        def _():
            pltpu.sync_copy(acc, out_hbm.at[cid])

    return scatter


def _combine_tc(n):
    def body(p_ref, c_ref, o_ref):
        p = p_ref[0] + p_ref[1]
        o_ref[...] = c_ref[...] + p[:, :3] / jnp.float32(100.0)

    return pl.pallas_call(
        body,
        out_shape=jax.ShapeDtypeStruct((n, 3), jnp.float32),
    )


def kernel(h, coord, edge_index, coord_diff, edge_attr, W1, b1, W2, b2, W3):
    n, hnf = h.shape
    e = edge_index.shape[1]
    f32 = jnp.float32
    row = edge_index[0].astype(jnp.int32)
    col = edge_index[1].astype(jnp.int32)
    w1t = W1.T.astype(f32)
    w1a = w1t[:hnf]
    w1b = w1t[hnf:2 * hnf]
    nea = w1t.shape[0] - 2 * hnf
    w1c = jnp.zeros((8, hnf), f32).at[:nea].set(w1t[2 * hnf:])
    ea8 = jnp.zeros((e, 8), f32).at[:, :nea].set(edge_attr.astype(f32))
    cd4 = jnp.zeros((e, 4), f32).at[:, :3].set(coord_diff.astype(f32))
    b1r = b1.reshape(1, hnf).astype(f32)
    b2r = b2.reshape(1, hnf).astype(f32)
    w2t = W2.T.astype(f32)
    w3r = W3.reshape(1, hnf).astype(f32)
    zeros_n4 = jnp.zeros((n, 4), f32)

    A, B = _precompute_tc(n, hnf, 2000)(h.astype(f32), w1a, w1b)
    S = _gather_add_sc(e, hnf, 80)(A, B, row, col)
    trans = _mlp_tc(e, hnf, 1280)(S, ea8, cd4, w1c, b1r, w2t, b2r, w3r)
    row2d = row.reshape(e // 100, 100)
    parts = _scatter_sc(e, n, 100)(trans, row2d, zeros_n4)
    return _combine_tc(n)(parts, coord.astype(f32))


# speedup vs baseline: 3.9210x; 1.5474x over previous
import functools

import jax
import jax.numpy as jnp
from jax import lax
from jax.experimental import pallas as pl
from jax.experimental.pallas import tpu as pltpu
from jax.experimental.pallas import tpu_sc as plsc

NC = 2
NS = 16
NW = NC * NS


def _precompute_tc(n, hnf, bn):
    def body(h_ref, wa_ref, wb_ref, a_ref, b_ref):
        hb = h_ref[...]
        a_ref[...] = jnp.dot(hb, wa_ref[...], preferred_element_type=jnp.float32)
        b_ref[...] = jnp.dot(hb, wb_ref[...], preferred_element_type=jnp.float32)

    return pl.pallas_call(
        body,
        grid=(n // bn,),
        in_specs=[
            pl.BlockSpec((bn, hnf), lambda i: (i, 0)),
            pl.BlockSpec((hnf, hnf), lambda i: (0, 0)),
            pl.BlockSpec((hnf, hnf), lambda i: (0, 0)),
        ],
        out_specs=[
            pl.BlockSpec((bn, hnf), lambda i: (i, 0)),
            pl.BlockSpec((bn, hnf), lambda i: (i, 0)),
        ],
        out_shape=[
            jax.ShapeDtypeStruct((n, hnf), jnp.float32),
            jax.ShapeDtypeStruct((n, hnf), jnp.float32),
        ],
    )


def _gather_add_sc(e, hnf, k):
    ew = e // NW
    nch = ew // k
    mesh = plsc.VectorSubcoreMesh(core_axis_name="c", subcore_axis_name="s")

    @functools.partial(
        pl.kernel,
        out_type=jax.ShapeDtypeStruct((e, hnf), jnp.float32),
        mesh=mesh,
        scratch_types=[
            pltpu.VMEM((ew,), jnp.int32),
            pltpu.VMEM((ew,), jnp.int32),
            pltpu.VMEM((k, hnf), jnp.float32),
            pltpu.VMEM((k, hnf), jnp.float32),
            pltpu.VMEM((k, hnf), jnp.float32),
            pltpu.VMEM((k, hnf), jnp.float32),
            pltpu.VMEM((k, hnf), jnp.float32),
            pltpu.VMEM((k, hnf), jnp.float32),
            pltpu.SemaphoreType.DMA,
            pltpu.SemaphoreType.DMA,
            pltpu.SemaphoreType.DMA,
            pltpu.SemaphoreType.DMA,
            pltpu.SemaphoreType.DMA,
            pltpu.SemaphoreType.DMA,
        ],
    )
    def gather_add(a_hbm, b_hbm, row_hbm, col_hbm, s_hbm,
                   idr, idc, ba0, ba1, bb0, bb1, sb0, sb1,
                   sma0, sma1, smb0, smb1, smw0, smw1):
        cid = lax.axis_index("c")
        sid = lax.axis_index("s")
        wid = sid * NC + cid
        base = wid * ew
        pltpu.sync_copy(row_hbm.at[pl.ds(base, ew)], idr)
        pltpu.sync_copy(col_hbm.at[pl.ds(base, ew)], idc)
        ba = (ba0, ba1)
        bb = (bb0, bb1)
        sb = (sb0, sb1)
        sma = (sma0, sma1)
        smb = (smb0, smb1)
        smw = (smw0, smw1)

        def issue(g, b):
            pltpu.make_async_copy(
                a_hbm.at[idr.at[pl.ds(g * k, k)]], ba[b], sma[b]).start()
            pltpu.make_async_copy(
                b_hbm.at[idc.at[pl.ds(g * k, k)]], bb[b], smb[b]).start()

        def addrows(b):
            def addrow(r, c2):
                for tt in range(hnf // 16):
                    sl = pl.ds(tt * 16, 16)
                    sb[b][r, sl] = ba[b][r, sl] + bb[b][r, sl]
                return c2

            lax.fori_loop(0, k, addrow, 0)

        def step(g, b):
            @pl.when(g + 1 < nch)
            def _():
                issue(g + 1, 1 - b)

            pltpu.make_async_copy(
                a_hbm.at[idr.at[pl.ds(g * k, k)]], ba[b], sma[b]).wait()
            pltpu.make_async_copy(
                b_hbm.at[idc.at[pl.ds(g * k, k)]], bb[b], smb[b]).wait()

            @pl.when(g >= 2)
            def _():
                pltpu.make_async_copy(
                    sb[b], s_hbm.at[pl.ds(base, k)], smw[b]).wait()

            addrows(b)
            pltpu.make_async_copy(
                sb[b], s_hbm.at[pl.ds(base + g * k, k)], smw[b]).start()

        issue(0, 0)

        def pair(t, carry):
            for b in range(2):
                step(t * 2 + b, b)
            return carry

        lax.fori_loop(0, nch // 2, pair, 0)
        if nch % 2:
            step(nch - 1, 0)
        pltpu.make_async_copy(sb0, s_hbm.at[pl.ds(base, k)], smw0).wait()
        pltpu.make_async_copy(sb1, s_hbm.at[pl.ds(base, k)], smw1).wait()

    return gather_add


def _mlp_tc(e, hnf, be):
    def body(s_ref, ea_ref, cd_ref, w1c_ref, b1_ref, w2_ref, b2_ref, w3_ref,
             o_ref):
        eaw = lax.dot_general(ea_ref[...], w1c_ref[...],
                              (((0,), (0,)), ((), ())),
                              preferred_element_type=jnp.float32)
        u = s_ref[...] + eaw + b1_ref[...]
        u = u * lax.logistic(u)
        x = jnp.dot(u, w2_ref[...], preferred_element_type=jnp.float32) + b2_ref[...]
        x = x * lax.logistic(x)
        m = lax.dot_general(w3_ref[...], x, (((1,), (1,)), ((), ())),
                            preferred_element_type=jnp.float32)
        o_ref[...] = cd_ref[...] * m

    return pl.pallas_call(
        body,
        grid=(e // be,),
        in_specs=[
            pl.BlockSpec((be, hnf), lambda i: (i, 0)),
            pl.BlockSpec((3, be), lambda i: (0, i)),
            pl.BlockSpec((4, be), lambda i: (0, i)),
            pl.BlockSpec((3, hnf), lambda i: (0, 0)),
            pl.BlockSpec((1, hnf), lambda i: (0, 0)),
            pl.BlockSpec((hnf, hnf), lambda i: (0, 0)),
            pl.BlockSpec((1, hnf), lambda i: (0, 0)),
            pl.BlockSpec((1, hnf), lambda i: (0, 0)),
        ],
        out_specs=pl.BlockSpec((4, be), lambda i: (0, i)),
        out_shape=jax.ShapeDtypeStruct((4, e), jnp.float32),
    )


def _scatter_sc(e, n, k):
    ew = e // NW
    nch = ew // k
    mesh = plsc.VectorSubcoreMesh(core_axis_name="c", subcore_axis_name="s")

    @functools.partial(
        pl.kernel,
        out_type=jax.ShapeDtypeStruct((NC, n, 4), jnp.float32),
        mesh=mesh,
        compiler_params=pltpu.CompilerParams(use_tc_tiling_on_sc=False),
        scratch_types=[
            pltpu.VMEM((nch, k), jnp.int32),
            pltpu.VMEM((ew, 4), jnp.float32),
            pltpu.VMEM_SHARED((n, 4), jnp.float32),
        ],
    )
    def scatter(trans_hbm, row2d_hbm, zero_hbm, out_hbm, idx2, tbuf, acc):
        cid = lax.axis_index("c")
        sid = lax.axis_index("s")
        wid = sid * NC + cid

        @pl.when(sid == 0)
        def _():
            pltpu.sync_copy(zero_hbm, acc)

        pltpu.sync_copy(row2d_hbm.at[pl.ds(wid * nch, nch)], idx2)
        pltpu.sync_copy(trans_hbm.at[pl.ds(wid * ew, ew)], tbuf)
        plsc.subcore_barrier()

        def chunk(j, carry):
            pltpu.sync_copy(tbuf.at[pl.ds(j * k, k)], acc.at[idx2.at[j]],
                            add=True)
            return carry

        lax.fori_loop(0, nch, chunk, 0)
        plsc.subcore_barrier()

        @pl.when(sid == 0)
        def _():
            pltpu.sync_copy(acc, out_hbm.at[cid])

    return scatter


def _combine_tc(n):
    def body(p_ref, c_ref, o_ref):
        p = p_ref[0] + p_ref[1]
        o_ref[...] = c_ref[...] + p[:, :3] / jnp.float32(100.0)

    return pl.pallas_call(
        body,
        out_shape=jax.ShapeDtypeStruct((n, 3), jnp.float32),
    )


def kernel(h, coord, edge_index, coord_diff, edge_attr, W1, b1, W2, b2, W3):
    n, hnf = h.shape
    e = edge_index.shape[1]
    f32 = jnp.float32
    row = edge_index[0].astype(jnp.int32)
    col = edge_index[1].astype(jnp.int32)
    w1t = W1.T.astype(f32)
    w1a = w1t[:hnf]
    w1b = w1t[hnf:2 * hnf]
    w1c = w1t[2 * hnf:]
    ea_t = edge_attr.astype(f32).T
    cd_t = jnp.concatenate(
        [coord_diff.astype(f32).T, jnp.zeros((1, e), f32)], axis=0)
    b1r = b1.reshape(1, hnf).astype(f32)
    b2r = b2.reshape(1, hnf).astype(f32)
    w2t = W2.T.astype(f32)
    w3r = W3.reshape(1, hnf).astype(f32)
    zeros_n4 = jnp.zeros((n, 4), f32)

    A, B = _precompute_tc(n, hnf, 2000)(h.astype(f32), w1a, w1b)
    S = _gather_add_sc(e, hnf, 80)(A, B, row, col)
    trans = _mlp_tc(e, hnf, 1280)(S, ea_t, cd_t, w1c, b1r, w2t, b2r, w3r)
    row2d = row.reshape(e // 100, 100)
    parts = _scatter_sc(e, n, 100)(trans.T, row2d, zeros_n4)
    return _combine_tc(n)(parts, coord.astype(f32))


# speedup vs baseline: 3.9252x; 1.0011x over previous
import functools

import jax
import jax.numpy as jnp
from jax import lax
from jax.experimental import pallas as pl
from jax.experimental.pallas import tpu as pltpu
from jax.experimental.pallas import tpu_sc as plsc

NC = 2
NS = 16
NW = NC * NS


def _precompute_tc(n, hnf, bn):
    def body(h_ref, wa_ref, wb_ref, a_ref, b_ref):
        hb = h_ref[...]
        a_ref[...] = jnp.dot(hb, wa_ref[...], preferred_element_type=jnp.float32)
        b_ref[...] = jnp.dot(hb, wb_ref[...], preferred_element_type=jnp.float32)

    return pl.pallas_call(
        body,
        grid=(n // bn,),
        in_specs=[
            pl.BlockSpec((bn, hnf), lambda i: (i, 0)),
            pl.BlockSpec((hnf, hnf), lambda i: (0, 0)),
            pl.BlockSpec((hnf, hnf), lambda i: (0, 0)),
        ],
        out_specs=[
            pl.BlockSpec((bn, hnf), lambda i: (i, 0)),
            pl.BlockSpec((bn, hnf), lambda i: (i, 0)),
        ],
        out_shape=[
            jax.ShapeDtypeStruct((n, hnf), jnp.float32),
            jax.ShapeDtypeStruct((n, hnf), jnp.float32),
        ],
    )


def _gather_add_sc(e, hnf, k):
    ew = e // NW
    nch = ew // k
    mesh = plsc.VectorSubcoreMesh(core_axis_name="c", subcore_axis_name="s")

    @functools.partial(
        pl.kernel,
        out_type=jax.ShapeDtypeStruct((e, hnf), jnp.float32),
        mesh=mesh,
        scratch_types=[
            pltpu.VMEM((ew,), jnp.int32),
            pltpu.VMEM((ew,), jnp.int32),
            pltpu.VMEM((k, hnf), jnp.float32),
            pltpu.VMEM((k, hnf), jnp.float32),
            pltpu.VMEM((k, hnf), jnp.float32),
            pltpu.VMEM((k, hnf), jnp.float32),
            pltpu.VMEM((k, hnf), jnp.float32),
            pltpu.VMEM((k, hnf), jnp.float32),
            pltpu.SemaphoreType.DMA,
            pltpu.SemaphoreType.DMA,
            pltpu.SemaphoreType.DMA,
            pltpu.SemaphoreType.DMA,
            pltpu.SemaphoreType.DMA,
            pltpu.SemaphoreType.DMA,
        ],
    )
    def gather_add(a_hbm, b_hbm, row_hbm, col_hbm, s_hbm,
                   idr, idc, ba0, ba1, bb0, bb1, sb0, sb1,
                   sma0, sma1, smb0, smb1, smw0, smw1):
        cid = lax.axis_index("c")
        sid = lax.axis_index("s")
        wid = sid * NC + cid
        base = wid * ew
        pltpu.sync_copy(row_hbm.at[pl.ds(base, ew)], idr)
        pltpu.sync_copy(col_hbm.at[pl.ds(base, ew)], idc)
        ba = (ba0, ba1)
        bb = (bb0, bb1)
        sb = (sb0, sb1)
        sma = (sma0, sma1)
        smb = (smb0, smb1)
        smw = (smw0, smw1)

        def issue(g, b):
            pltpu.make_async_copy(
                a_hbm.at[idr.at[pl.ds(g * k, k)]], ba[b], sma[b]).start()
            pltpu.make_async_copy(
                b_hbm.at[idc.at[pl.ds(g * k, k)]], bb[b], smb[b]).start()

        def addrows(b):
            def addrow(r, c2):
                for tt in range(hnf // 16):
                    sl = pl.ds(tt * 16, 16)
                    sb[b][r, sl] = ba[b][r, sl] + bb[b][r, sl]
                return c2

            lax.fori_loop(0, k, addrow, 0)

        def step(g, b):
            @pl.when(g + 1 < nch)
            def _():
                issue(g + 1, 1 - b)

            pltpu.make_async_copy(
                a_hbm.at[idr.at[pl.ds(g * k, k)]], ba[b], sma[b]).wait()
            pltpu.make_async_copy(
                b_hbm.at[idc.at[pl.ds(g * k, k)]], bb[b], smb[b]).wait()

            @pl.when(g >= 2)
            def _():
                pltpu.make_async_copy(
                    sb[b], s_hbm.at[pl.ds(base, k)], smw[b]).wait()

            addrows(b)
            pltpu.make_async_copy(
                sb[b], s_hbm.at[pl.ds(base + g * k, k)], smw[b]).start()

        issue(0, 0)

        def pair(t, carry):
            for b in range(2):
                step(t * 2 + b, b)
            return carry

        lax.fori_loop(0, nch // 2, pair, 0)
        if nch % 2:
            step(nch - 1, 0)
        pltpu.make_async_copy(sb0, s_hbm.at[pl.ds(base, k)], smw0).wait()
        pltpu.make_async_copy(sb1, s_hbm.at[pl.ds(base, k)], smw1).wait()

    return gather_add


def _mlp_tc(e, hnf, be):
    def body(s_ref, ea_ref, cd_ref, w1c_ref, b1_ref, w2_ref, b2_ref, w3_ref,
             o_ref):
        eaw = lax.dot_general(ea_ref[...], w1c_ref[...],
                              (((0,), (0,)), ((), ())),
                              preferred_element_type=jnp.float32)
        u = s_ref[...] + eaw + b1_ref[...]
        u = u * lax.logistic(u)
        x = jnp.dot(u.astype(jnp.bfloat16), w2_ref[...].astype(jnp.bfloat16),
                    preferred_element_type=jnp.float32) + b2_ref[...]
        x = x * lax.logistic(x)
        m = lax.dot_general(w3_ref[...].astype(jnp.bfloat16),
                            x.astype(jnp.bfloat16), (((1,), (1,)), ((), ())),
                            preferred_element_type=jnp.float32)
        o_ref[...] = cd_ref[...] * m

    return pl.pallas_call(
        body,
        grid=(e // be,),
        in_specs=[
            pl.BlockSpec((be, hnf), lambda i: (i, 0)),
            pl.BlockSpec((3, be), lambda i: (0, i)),
            pl.BlockSpec((4, be), lambda i: (0, i)),
            pl.BlockSpec((3, hnf), lambda i: (0, 0)),
            pl.BlockSpec((1, hnf), lambda i: (0, 0)),
            pl.BlockSpec((hnf, hnf), lambda i: (0, 0)),
            pl.BlockSpec((1, hnf), lambda i: (0, 0)),
            pl.BlockSpec((1, hnf), lambda i: (0, 0)),
        ],
        out_specs=pl.BlockSpec((4, be), lambda i: (0, i)),
        out_shape=jax.ShapeDtypeStruct((4, e), jnp.float32),
    )


def _scatter_sc(e, n, k):
    ew = e // NW
    nch = ew // k
    mesh = plsc.VectorSubcoreMesh(core_axis_name="c", subcore_axis_name="s")

    @functools.partial(
        pl.kernel,
        out_type=jax.ShapeDtypeStruct((NC, n, 4), jnp.float32),
        mesh=mesh,
        compiler_params=pltpu.CompilerParams(use_tc_tiling_on_sc=False),
        scratch_types=[
            pltpu.VMEM((nch, k), jnp.int32),
            pltpu.VMEM((ew, 4), jnp.float32),
            pltpu.VMEM_SHARED((n, 4), jnp.float32),
        ],
    )
    def scatter(trans_hbm, row2d_hbm, zero_hbm, out_hbm, idx2, tbuf, acc):
        cid = lax.axis_index("c")
        sid = lax.axis_index("s")
        wid = sid * NC + cid

        @pl.when(sid == 0)
        def _():
            pltpu.sync_copy(zero_hbm, acc)

        pltpu.sync_copy(row2d_hbm.at[pl.ds(wid * nch, nch)], idx2)
        pltpu.sync_copy(trans_hbm.at[pl.ds(wid * ew, ew)], tbuf)
        plsc.subcore_barrier()

        def chunk(j, carry):
            pltpu.sync_copy(tbuf.at[pl.ds(j * k, k)], acc.at[idx2.at[j]],
                            add=True)
            return carry

        lax.fori_loop(0, nch, chunk, 0)
        plsc.subcore_barrier()

        @pl.when(sid == 0)
        def _():
            pltpu.sync_copy(acc, out_hbm.at[cid])

    return scatter


def _combine_tc(n):
    def body(p_ref, c_ref, o_ref):
        p = p_ref[0] + p_ref[1]
        o_ref[...] = c_ref[...] + p[:, :3] / jnp.float32(100.0)

    return pl.pallas_call(
        body,
        out_shape=jax.ShapeDtypeStruct((n, 3), jnp.float32),
    )


def kernel(h, coord, edge_index, coord_diff, edge_attr, W1, b1, W2, b2, W3):
    n, hnf = h.shape
    e = edge_index.shape[1]
    f32 = jnp.float32
    row = edge_index[0].astype(jnp.int32)
    col = edge_index[1].astype(jnp.int32)
    w1t = W1.T.astype(f32)
    w1a = w1t[:hnf]
    w1b = w1t[hnf:2 * hnf]
    w1c = w1t[2 * hnf:]
    ea_t = edge_attr.astype(f32).T
    cd_t = jnp.concatenate(
        [coord_diff.astype(f32).T, jnp.zeros((1, e), f32)], axis=0)
    b1r = b1.reshape(1, hnf).astype(f32)
    b2r = b2.reshape(1, hnf).astype(f32)
    w2t = W2.T.astype(f32)
    w3r = W3.reshape(1, hnf).astype(f32)
    zeros_n4 = jnp.zeros((n, 4), f32)

    A, B = _precompute_tc(n, hnf, 2000)(h.astype(f32), w1a, w1b)
    S = _gather_add_sc(e, hnf, 80)(A, B, row, col)
    trans = _mlp_tc(e, hnf, 1280)(S, ea_t, cd_t, w1c, b1r, w2t, b2r, w3r)
    row2d = row.reshape(e // 100, 100)
    parts = _scatter_sc(e, n, 100)(trans.T, row2d, zeros_n4)
    return _combine_tc(n)(parts, coord.astype(f32))


# speedup vs baseline: 6.1068x; 1.5558x over previous
import functools

import jax
import jax.numpy as jnp
from jax import lax
from jax.experimental import pallas as pl
from jax.experimental.pallas import tpu as pltpu
from jax.experimental.pallas import tpu_sc as plsc

NC = 2
NS = 16
NW = NC * NS


def _precompute_tc(n, hnf, bn):
    def body(h_ref, wa_ref, wb_ref, a_ref, b_ref):
        hb = h_ref[...]
        a_ref[...] = jnp.dot(hb, wa_ref[...], preferred_element_type=jnp.float32)
        b_ref[...] = jnp.dot(hb, wb_ref[...], preferred_element_type=jnp.float32)

    return pl.pallas_call(
        body,
        grid=(n // bn,),
        in_specs=[
            pl.BlockSpec((bn, hnf), lambda i: (i, 0)),
            pl.BlockSpec((hnf, hnf), lambda i: (0, 0)),
            pl.BlockSpec((hnf, hnf), lambda i: (0, 0)),
        ],
        out_specs=[
            pl.BlockSpec((bn, hnf), lambda i: (i, 0)),
            pl.BlockSpec((bn, hnf), lambda i: (i, 0)),
        ],
        out_shape=[
            jax.ShapeDtypeStruct((n, hnf), jnp.float32),
            jax.ShapeDtypeStruct((n, hnf), jnp.float32),
        ],
    )


def _gather_add_sc(e, hnf, k):
    ew = e // NW
    nch = ew // k
    mesh = plsc.VectorSubcoreMesh(core_axis_name="c", subcore_axis_name="s")

    @functools.partial(
        pl.kernel,
        out_type=jax.ShapeDtypeStruct((e, hnf), jnp.float32),
        mesh=mesh,
        scratch_types=[
            pltpu.VMEM((ew,), jnp.int32),
            pltpu.VMEM((ew,), jnp.int32),
            pltpu.VMEM((k, hnf), jnp.float32),
            pltpu.VMEM((k, hnf), jnp.float32),
            pltpu.VMEM((k, hnf), jnp.float32),
            pltpu.VMEM((k, hnf), jnp.float32),
            pltpu.VMEM((k, hnf), jnp.float32),
            pltpu.VMEM((k, hnf), jnp.float32),
            pltpu.SemaphoreType.DMA,
            pltpu.SemaphoreType.DMA,
            pltpu.SemaphoreType.DMA,
            pltpu.SemaphoreType.DMA,
            pltpu.SemaphoreType.DMA,
            pltpu.SemaphoreType.DMA,
        ],
    )
    def gather_add(a_hbm, b_hbm, row_hbm, col_hbm, s_hbm,
                   idr, idc, ba0, ba1, bb0, bb1, sb0, sb1,
                   sma0, sma1, smb0, smb1, smw0, smw1):
        cid = lax.axis_index("c")
        sid = lax.axis_index("s")
        wid = sid * NC + cid
        base = wid * ew
        pltpu.sync_copy(row_hbm.at[pl.ds(base, ew)], idr)
        pltpu.sync_copy(col_hbm.at[pl.ds(base, ew)], idc)
        ba = (ba0, ba1)
        bb = (bb0, bb1)
        sb = (sb0, sb1)
        sma = (sma0, sma1)
        smb = (smb0, smb1)
        smw = (smw0, smw1)

        def issue(g, b):
            pltpu.make_async_copy(
                a_hbm.at[idr.at[pl.ds(g * k, k)]], ba[b], sma[b]).start()
            pltpu.make_async_copy(
                b_hbm.at[idc.at[pl.ds(g * k, k)]], bb[b], smb[b]).start()

        def addrows(b):
            def addrow(r, c2):
                for tt in range(hnf // 16):
                    sl = pl.ds(tt * 16, 16)
                    sb[b][r, sl] = ba[b][r, sl] + bb[b][r, sl]
                return c2

            lax.fori_loop(0, k, addrow, 0)

        def step(g, b):
            @pl.when(g + 1 < nch)
            def _():
                issue(g + 1, 1 - b)

            pltpu.make_async_copy(
                a_hbm.at[idr.at[pl.ds(g * k, k)]], ba[b], sma[b]).wait()
            pltpu.make_async_copy(
                b_hbm.at[idc.at[pl.ds(g * k, k)]], bb[b], smb[b]).wait()

            @pl.when(g >= 2)
            def _():
                pltpu.make_async_copy(
                    sb[b], s_hbm.at[pl.ds(base, k)], smw[b]).wait()

            addrows(b)
            pltpu.make_async_copy(
                sb[b], s_hbm.at[pl.ds(base + g * k, k)], smw[b]).start()

        issue(0, 0)

        def pair(t, carry):
            for b in range(2):
                step(t * 2 + b, b)
            return carry

        lax.fori_loop(0, nch // 2, pair, 0)
        if nch % 2:
            step(nch - 1, 0)
        pltpu.make_async_copy(sb0, s_hbm.at[pl.ds(base, k)], smw0).wait()
        pltpu.make_async_copy(sb1, s_hbm.at[pl.ds(base, k)], smw1).wait()

    return gather_add


def _mlp_tc(e, hnf, be):
    def body(s_ref, ea_ref, cd_ref, w1c_ref, b1_ref, w2_ref, b2_ref, w3_ref,
             o_ref):
        eaw = lax.dot_general(ea_ref[...], w1c_ref[...],
                              (((0,), (0,)), ((), ())),
                              preferred_element_type=jnp.float32)
        u = s_ref[...] + eaw + b1_ref[...]
        u = u * lax.logistic(u)
        x = jnp.dot(u.astype(jnp.bfloat16), w2_ref[...].astype(jnp.bfloat16),
                    preferred_element_type=jnp.float32) + b2_ref[...]
        x = x * lax.logistic(x)
        m = lax.dot_general(w3_ref[...].astype(jnp.bfloat16),
                            x.astype(jnp.bfloat16), (((1,), (1,)), ((), ())),
                            preferred_element_type=jnp.float32)
        o_ref[...] = cd_ref[...] * m

    return pl.pallas_call(
        body,
        grid=(e // be,),
        in_specs=[
            pl.BlockSpec((be, hnf), lambda i: (i, 0)),
            pl.BlockSpec((3, be), lambda i: (0, i)),
            pl.BlockSpec((4, be), lambda i: (0, i)),
            pl.BlockSpec((3, hnf), lambda i: (0, 0)),
            pl.BlockSpec((1, hnf), lambda i: (0, 0)),
            pl.BlockSpec((hnf, hnf), lambda i: (0, 0)),
            pl.BlockSpec((1, hnf), lambda i: (0, 0)),
            pl.BlockSpec((1, hnf), lambda i: (0, 0)),
        ],
        out_specs=pl.BlockSpec((4, be), lambda i: (0, i)),
        out_shape=jax.ShapeDtypeStruct((4, e), jnp.float32),
    )


def _scatter_sc(e, n, k):
    ew = e // NW
    nch = ew // k
    mesh = plsc.VectorSubcoreMesh(core_axis_name="c", subcore_axis_name="s")

    @functools.partial(
        pl.kernel,
        out_type=jax.ShapeDtypeStruct((NC, 3, n), jnp.float32),
        mesh=mesh,
        compiler_params=pltpu.CompilerParams(use_tc_tiling_on_sc=False),
        scratch_types=[
            pltpu.VMEM((nch, k), jnp.int32),
            pltpu.VMEM((3, ew), jnp.float32),
            pltpu.VMEM_SHARED((3, n), jnp.float32),
        ],
    )
    def scatter(trans_hbm, row2d_hbm, zero_hbm, out_hbm, idx2, tbuf, acc):
        cid = lax.axis_index("c")
        sid = lax.axis_index("s")
        wid = sid * NC + cid

        @pl.when(sid == 0)
        def _():
            pltpu.sync_copy(zero_hbm, acc)

        pltpu.sync_copy(row2d_hbm.at[pl.ds(wid * nch, nch)], idx2)
        for d in range(3):
            pltpu.sync_copy(trans_hbm.at[d, pl.ds(wid * ew, ew)], tbuf.at[d])
        plsc.subcore_barrier()

        def chunk(j, carry):
            for d in range(3):
                pltpu.sync_copy(tbuf.at[d, pl.ds(j * k, k)],
                                acc.at[d].at[idx2.at[j]], add=True)
            return carry

        lax.fori_loop(0, nch, chunk, 0)
        plsc.subcore_barrier()

        @pl.when(sid == 0)
        def _():
            pltpu.sync_copy(acc, out_hbm.at[cid])

    return scatter


def _combine_tc(n):
    def body(p_ref, c_ref, o_ref):
        p = p_ref[0] + p_ref[1]
        o_ref[...] = c_ref[...] + p.T / jnp.float32(100.0)

    return pl.pallas_call(
        body,
        out_shape=jax.ShapeDtypeStruct((n, 3), jnp.float32),
    )


def kernel(h, coord, edge_index, coord_diff, edge_attr, W1, b1, W2, b2, W3):
    n, hnf = h.shape
    e = edge_index.shape[1]
    f32 = jnp.float32
    row = edge_index[0].astype(jnp.int32)
    col = edge_index[1].astype(jnp.int32)
    w1t = W1.T.astype(f32)
    w1a = w1t[:hnf]
    w1b = w1t[hnf:2 * hnf]
    w1c = w1t[2 * hnf:]
    ea_t = edge_attr.astype(f32).T
    cd_t = jnp.concatenate(
        [coord_diff.astype(f32).T, jnp.zeros((1, e), f32)], axis=0)
    b1r = b1.reshape(1, hnf).astype(f32)
    b2r = b2.reshape(1, hnf).astype(f32)
    w2t = W2.T.astype(f32)
    w3r = W3.reshape(1, hnf).astype(f32)
    zeros_3n = jnp.zeros((3, n), f32)

    A, B = _precompute_tc(n, hnf, 2000)(h.astype(f32), w1a, w1b)
    S = _gather_add_sc(e, hnf, 80)(A, B, row, col)
    trans = _mlp_tc(e, hnf, 1280)(S, ea_t, cd_t, w1c, b1r, w2t, b2r, w3r)
    row2d = row.reshape(e // 200, 200)
    parts = _scatter_sc(e, n, 200)(trans, row2d, zeros_3n)
    return _combine_tc(n)(parts, coord.astype(f32))


# speedup vs baseline: 6.6392x; 1.0872x over previous
import functools

import jax
import jax.numpy as jnp
from jax import lax
from jax.experimental import pallas as pl
from jax.experimental.pallas import tpu as pltpu
from jax.experimental.pallas import tpu_sc as plsc

NC = 2
NS = 16
NW = NC * NS


def _precompute_tc(n, hnf, bn):
    def body(h_ref, wa_ref, wb_ref, a_ref, b_ref):
        hb = h_ref[...]
        a_ref[...] = jnp.dot(hb, wa_ref[...], preferred_element_type=jnp.float32)
        b_ref[...] = jnp.dot(hb, wb_ref[...], preferred_element_type=jnp.float32)

    return pl.pallas_call(
        body,
        grid=(n // bn,),
        in_specs=[
            pl.BlockSpec((bn, hnf), lambda i: (i, 0)),
            pl.BlockSpec((hnf, hnf), lambda i: (0, 0)),
            pl.BlockSpec((hnf, hnf), lambda i: (0, 0)),
        ],
        out_specs=[
            pl.BlockSpec((bn, hnf), lambda i: (i, 0)),
            pl.BlockSpec((bn, hnf), lambda i: (i, 0)),
        ],
        out_shape=[
            jax.ShapeDtypeStruct((n, hnf), jnp.float32),
            jax.ShapeDtypeStruct((n, hnf), jnp.float32),
        ],
    )


def _gather_add_sc(e, hnf, k):
    ew = e // NW
    nch = ew // k
    mesh = plsc.VectorSubcoreMesh(core_axis_name="c", subcore_axis_name="s")

    @functools.partial(
        pl.kernel,
        out_type=jax.ShapeDtypeStruct((e, hnf), jnp.float32),
        mesh=mesh,
        scratch_types=[
            pltpu.VMEM((ew,), jnp.int32),
            pltpu.VMEM((ew,), jnp.int32),
            pltpu.VMEM((k, hnf), jnp.float32),
            pltpu.VMEM((k, hnf), jnp.float32),
            pltpu.VMEM((k, hnf), jnp.float32),
            pltpu.VMEM((k, hnf), jnp.float32),
            pltpu.VMEM((k, hnf), jnp.float32),
            pltpu.VMEM((k, hnf), jnp.float32),
            pltpu.SemaphoreType.DMA,
            pltpu.SemaphoreType.DMA,
            pltpu.SemaphoreType.DMA,
            pltpu.SemaphoreType.DMA,
            pltpu.SemaphoreType.DMA,
            pltpu.SemaphoreType.DMA,
        ],
    )
    def gather_add(a_hbm, b_hbm, row_hbm, col_hbm, s_hbm,
                   idr, idc, ba0, ba1, bb0, bb1, sb0, sb1,
                   sma0, sma1, smb0, smb1, smw0, smw1):
        cid = lax.axis_index("c")
        sid = lax.axis_index("s")
        wid = sid * NC + cid
        base = wid * ew
        pltpu.sync_copy(row_hbm.at[pl.ds(base, ew)], idr)
        pltpu.sync_copy(col_hbm.at[pl.ds(base, ew)], idc)
        ba = (ba0, ba1)
        bb = (bb0, bb1)
        sb = (sb0, sb1)
        sma = (sma0, sma1)
        smb = (smb0, smb1)
        smw = (smw0, smw1)

        def issue(g, b):
            pltpu.make_async_copy(
                a_hbm.at[idr.at[pl.ds(g * k, k)]], ba[b], sma[b]).start()
            pltpu.make_async_copy(
                b_hbm.at[idc.at[pl.ds(g * k, k)]], bb[b], smb[b]).start()

        def addrows(b):
            def addrow(r, c2):
                for tt in range(hnf // 16):
                    sl = pl.ds(tt * 16, 16)
                    sb[b][r, sl] = ba[b][r, sl] + bb[b][r, sl]
                return c2

            lax.fori_loop(0, k, addrow, 0)

        def step(g, b):
            @pl.when(g + 1 < nch)
            def _():
                issue(g + 1, 1 - b)

            pltpu.make_async_copy(
                a_hbm.at[idr.at[pl.ds(g * k, k)]], ba[b], sma[b]).wait()
            pltpu.make_async_copy(
                b_hbm.at[idc.at[pl.ds(g * k, k)]], bb[b], smb[b]).wait()

            @pl.when(g >= 2)
            def _():
                pltpu.make_async_copy(
                    sb[b], s_hbm.at[pl.ds(base, k)], smw[b]).wait()

            addrows(b)
            pltpu.make_async_copy(
                sb[b], s_hbm.at[pl.ds(base + g * k, k)], smw[b]).start()

        issue(0, 0)

        def pair(t, carry):
            for b in range(2):
                step(t * 2 + b, b)
            return carry

        lax.fori_loop(0, nch // 2, pair, 0)
        if nch % 2:
            step(nch - 1, 0)
        pltpu.make_async_copy(sb0, s_hbm.at[pl.ds(base, k)], smw0).wait()
        pltpu.make_async_copy(sb1, s_hbm.at[pl.ds(base, k)], smw1).wait()

    return gather_add


def _mlp_tc(e, hnf, be):
    def body(s_ref, ea_ref, cd_ref, w1c_ref, b1_ref, w2_ref, b2_ref, w3_ref,
             o_ref):
        eaw = lax.dot_general(ea_ref[...], w1c_ref[...],
                              (((0,), (0,)), ((), ())),
                              preferred_element_type=jnp.float32)
        u = s_ref[...] + eaw + b1_ref[...]
        u = u * lax.logistic(u)
        x = jnp.dot(u.astype(jnp.bfloat16), w2_ref[...].astype(jnp.bfloat16),
                    preferred_element_type=jnp.float32) + b2_ref[...]
        x = x * lax.logistic(x)
        m = lax.dot_general(w3_ref[...].astype(jnp.bfloat16),
                            x.astype(jnp.bfloat16), (((1,), (1,)), ((), ())),
                            preferred_element_type=jnp.float32)
        o_ref[...] = cd_ref[...] * m

    return pl.pallas_call(
        body,
        grid=(e // be,),
        in_specs=[
            pl.BlockSpec((be, hnf), lambda i: (i, 0)),
            pl.BlockSpec((3, be), lambda i: (0, i)),
            pl.BlockSpec((4, be), lambda i: (0, i)),
            pl.BlockSpec((3, hnf), lambda i: (0, 0)),
            pl.BlockSpec((1, hnf), lambda i: (0, 0)),
            pl.BlockSpec((hnf, hnf), lambda i: (0, 0)),
            pl.BlockSpec((1, hnf), lambda i: (0, 0)),
            pl.BlockSpec((1, hnf), lambda i: (0, 0)),
        ],
        out_specs=pl.BlockSpec((4, be), lambda i: (0, i)),
        out_shape=jax.ShapeDtypeStruct((4, e), jnp.float32),
    )


def _scatter_sc(e, n, k):
    ew = e // NW
    nch = ew // k
    mesh = plsc.VectorSubcoreMesh(core_axis_name="c", subcore_axis_name="s")

    @functools.partial(
        pl.kernel,
        out_type=jax.ShapeDtypeStruct((NC, 3, n), jnp.float32),
        mesh=mesh,
        compiler_params=pltpu.CompilerParams(use_tc_tiling_on_sc=False),
        scratch_types=[
            pltpu.VMEM((nch, k), jnp.int32),
            pltpu.VMEM((3, ew), jnp.float32),
            pltpu.VMEM_SHARED((3, n), jnp.float32),
        ],
    )
    def scatter(t1_hbm, t2_hbm, row2d_hbm, zero_hbm, out_hbm, idx2, tbuf, acc):
        cid = lax.axis_index("c")
        sid = lax.axis_index("s")
        wid = sid * NC + cid

        @pl.when(sid == 0)
        def _():
            pltpu.sync_copy(zero_hbm, acc)

        pltpu.sync_copy(row2d_hbm.at[pl.ds(wid * nch, nch)], idx2)
        half = NW // 2

        @pl.when(wid < half)
        def _():
            for d in range(3):
                pltpu.sync_copy(t1_hbm.at[d, pl.ds(wid * ew, ew)], tbuf.at[d])

        @pl.when(wid >= half)
        def _():
            for d in range(3):
                pltpu.sync_copy(t2_hbm.at[d, pl.ds((wid - half) * ew, ew)],
                                tbuf.at[d])
        plsc.subcore_barrier()

        def chunk(j, carry):
            for d in range(3):
                pltpu.sync_copy(tbuf.at[d, pl.ds(j * k, k)],
                                acc.at[d].at[idx2.at[j]], add=True)
            return carry

        lax.fori_loop(0, nch, chunk, 0)
        plsc.subcore_barrier()

        @pl.when(sid == 0)
        def _():
            pltpu.sync_copy(acc, out_hbm.at[cid])

    return scatter


def _combine_tc(n):
    def body(p_ref, c_ref, o_ref):
        p = p_ref[0] + p_ref[1]
        o_ref[...] = c_ref[...] + p.T / jnp.float32(100.0)

    return pl.pallas_call(
        body,
        out_shape=jax.ShapeDtypeStruct((n, 3), jnp.float32),
    )


def kernel(h, coord, edge_index, coord_diff, edge_attr, W1, b1, W2, b2, W3):
    n, hnf = h.shape
    e = edge_index.shape[1]
    f32 = jnp.float32
    row = edge_index[0].astype(jnp.int32)
    col = edge_index[1].astype(jnp.int32)
    w1t = W1.T.astype(f32)
    w1a = w1t[:hnf]
    w1b = w1t[hnf:2 * hnf]
    w1c = w1t[2 * hnf:]
    ea_t = edge_attr.astype(f32).T
    cd_t = jnp.concatenate(
        [coord_diff.astype(f32).T, jnp.zeros((1, e), f32)], axis=0)
    b1r = b1.reshape(1, hnf).astype(f32)
    b2r = b2.reshape(1, hnf).astype(f32)
    w2t = W2.T.astype(f32)
    w3r = W3.reshape(1, hnf).astype(f32)
    zeros_3n = jnp.zeros((3, n), f32)

    A, B = _precompute_tc(n, hnf, 2000)(h.astype(f32), w1a, w1b)
    he = e // 2
    gat = _gather_add_sc(he, hnf, 40)
    mlp = _mlp_tc(he, hnf, 1280)
    S1 = gat(A, B, row[:he], col[:he])
    S2 = gat(A, B, row[he:], col[he:])
    t1 = mlp(S1, ea_t[:, :he], cd_t[:, :he], w1c, b1r, w2t, b2r, w3r)
    t2 = mlp(S2, ea_t[:, he:], cd_t[:, he:], w1c, b1r, w2t, b2r, w3r)
    row2d = row.reshape(e // 200, 200)
    parts = _scatter_sc(e, n, 200)(t1, t2, row2d, zeros_3n)
    return _combine_tc(n)(parts, coord.astype(f32))


# speedup vs baseline: 6.7099x; 1.0106x over previous
import functools

import jax
import jax.numpy as jnp
from jax import lax
from jax.experimental import pallas as pl
from jax.experimental.pallas import tpu as pltpu
from jax.experimental.pallas import tpu_sc as plsc

NC = 2
NS = 16
NW = NC * NS


def _precompute_tc(n, hnf, bn):
    def body(h_ref, wa_ref, wb_ref, a_ref, b_ref):
        hb = h_ref[...]
        a_ref[...] = jnp.dot(hb, wa_ref[...], preferred_element_type=jnp.float32)
        b_ref[...] = jnp.dot(hb, wb_ref[...], preferred_element_type=jnp.float32)

    return pl.pallas_call(
        body,
        grid=(n // bn,),
        in_specs=[
            pl.BlockSpec((bn, hnf), lambda i: (i, 0)),
            pl.BlockSpec((hnf, hnf), lambda i: (0, 0)),
            pl.BlockSpec((hnf, hnf), lambda i: (0, 0)),
        ],
        out_specs=[
            pl.BlockSpec((bn, hnf), lambda i: (i, 0)),
            pl.BlockSpec((bn, hnf), lambda i: (i, 0)),
        ],
        out_shape=[
            jax.ShapeDtypeStruct((n, hnf), jnp.float32),
            jax.ShapeDtypeStruct((n, hnf), jnp.float32),
        ],
    )


def _gather_add_sc(e, hnf, k):
    ew = e // NW
    nch = ew // k
    mesh = plsc.VectorSubcoreMesh(core_axis_name="c", subcore_axis_name="s")

    @functools.partial(
        pl.kernel,
        out_type=jax.ShapeDtypeStruct((e, hnf), jnp.float32),
        mesh=mesh,
        scratch_types=[
            pltpu.VMEM((ew,), jnp.int32),
            pltpu.VMEM((ew,), jnp.int32),
            pltpu.VMEM((k, hnf), jnp.float32),
            pltpu.VMEM((k, hnf), jnp.float32),
            pltpu.VMEM((k, hnf), jnp.float32),
            pltpu.VMEM((k, hnf), jnp.float32),
            pltpu.VMEM((k, hnf), jnp.float32),
            pltpu.VMEM((k, hnf), jnp.float32),
            pltpu.SemaphoreType.DMA,
            pltpu.SemaphoreType.DMA,
            pltpu.SemaphoreType.DMA,
            pltpu.SemaphoreType.DMA,
            pltpu.SemaphoreType.DMA,
            pltpu.SemaphoreType.DMA,
        ],
    )
    def gather_add(a_hbm, b_hbm, row_hbm, col_hbm, s_hbm,
                   idr, idc, ba0, ba1, bb0, bb1, sb0, sb1,
                   sma0, sma1, smb0, smb1, smw0, smw1):
        cid = lax.axis_index("c")
        sid = lax.axis_index("s")
        wid = sid * NC + cid
        base = wid * ew
        pltpu.sync_copy(row_hbm.at[pl.ds(base, ew)], idr)
        pltpu.sync_copy(col_hbm.at[pl.ds(base, ew)], idc)
        ba = (ba0, ba1)
        bb = (bb0, bb1)
        sb = (sb0, sb1)
        sma = (sma0, sma1)
        smb = (smb0, smb1)
        smw = (smw0, smw1)

        def issue(g, b):
            pltpu.make_async_copy(
                a_hbm.at[idr.at[pl.ds(g * k, k)]], ba[b], sma[b]).start()
            pltpu.make_async_copy(
                b_hbm.at[idc.at[pl.ds(g * k, k)]], bb[b], smb[b]).start()

        def addrows(b):
            def addrow(r, c2):
                for tt in range(hnf // 16):
                    sl = pl.ds(tt * 16, 16)
                    sb[b][r, sl] = ba[b][r, sl] + bb[b][r, sl]
                return c2

            lax.fori_loop(0, k, addrow, 0)

        def step(g, b):
            @pl.when(g + 1 < nch)
            def _():
                issue(g + 1, 1 - b)

            pltpu.make_async_copy(
                a_hbm.at[idr.at[pl.ds(g * k, k)]], ba[b], sma[b]).wait()
            pltpu.make_async_copy(
                b_hbm.at[idc.at[pl.ds(g * k, k)]], bb[b], smb[b]).wait()

            @pl.when(g >= 2)
            def _():
                pltpu.make_async_copy(
                    sb[b], s_hbm.at[pl.ds(base, k)], smw[b]).wait()

            addrows(b)
            pltpu.make_async_copy(
                sb[b], s_hbm.at[pl.ds(base + g * k, k)], smw[b]).start()

        issue(0, 0)

        def pair(t, carry):
            for b in range(2):
                step(t * 2 + b, b)
            return carry

        lax.fori_loop(0, nch // 2, pair, 0)
        if nch % 2:
            step(nch - 1, 0)
        pltpu.make_async_copy(sb0, s_hbm.at[pl.ds(base, k)], smw0).wait()
        pltpu.make_async_copy(sb1, s_hbm.at[pl.ds(base, k)], smw1).wait()

    return gather_add


def _mlp_tc(e, hnf, be):
    def body(s_ref, ea_ref, cd_ref, w1c_ref, b1_ref, w2_ref, b2_ref, w3_ref,
             o_ref):
        eaw = lax.dot_general(ea_ref[...], w1c_ref[...],
                              (((0,), (0,)), ((), ())),
                              preferred_element_type=jnp.float32)
        u = s_ref[...] + eaw + b1_ref[...]
        u = u * lax.logistic(u)
        x = jnp.dot(u.astype(jnp.bfloat16), w2_ref[...].astype(jnp.bfloat16),
                    preferred_element_type=jnp.float32) + b2_ref[...]
        x = x * lax.logistic(x)
        m = lax.dot_general(w3_ref[...].astype(jnp.bfloat16),
                            x.astype(jnp.bfloat16), (((1,), (1,)), ((), ())),
                            preferred_element_type=jnp.float32)
        o_ref[...] = cd_ref[...] * m

    return pl.pallas_call(
        body,
        grid=(e // be,),
        in_specs=[
            pl.BlockSpec((be, hnf), lambda i: (i, 0)),
            pl.BlockSpec((3, be), lambda i: (0, i)),
            pl.BlockSpec((4, be), lambda i: (0, i)),
            pl.BlockSpec((3, hnf), lambda i: (0, 0)),
            pl.BlockSpec((1, hnf), lambda i: (0, 0)),
            pl.BlockSpec((hnf, hnf), lambda i: (0, 0)),
            pl.BlockSpec((1, hnf), lambda i: (0, 0)),
            pl.BlockSpec((1, hnf), lambda i: (0, 0)),
        ],
        out_specs=pl.BlockSpec((4, be), lambda i: (0, i)),
        out_shape=jax.ShapeDtypeStruct((4, e), jnp.float32),
    )


def _scatter_sc(ce, n, k):
    ew = ce // NW
    nch = ew // k
    mesh = plsc.VectorSubcoreMesh(core_axis_name="c", subcore_axis_name="s")

    @functools.partial(
        pl.kernel,
        out_type=jax.ShapeDtypeStruct((NC, 3, n), jnp.float32),
        mesh=mesh,
        compiler_params=pltpu.CompilerParams(use_tc_tiling_on_sc=False),
        scratch_types=[
            pltpu.VMEM((nch, k), jnp.int32),
            pltpu.VMEM((3, ew), jnp.float32),
            pltpu.VMEM_SHARED((3, n), jnp.float32),
        ],
    )
    def scatter(trans_hbm, row2d_hbm, zero_hbm, out_hbm, idx2, tbuf, acc):
        cid = lax.axis_index("c")
        sid = lax.axis_index("s")
        wid = sid * NC + cid

        @pl.when(sid == 0)
        def _():
            pltpu.sync_copy(zero_hbm, acc)

        pltpu.sync_copy(row2d_hbm.at[pl.ds(wid * nch, nch)], idx2)
        for d in range(3):
            pltpu.sync_copy(trans_hbm.at[d, pl.ds(wid * ew, ew)], tbuf.at[d])
        plsc.subcore_barrier()

        def chunk(j, carry):
            for d in range(3):
                pltpu.sync_copy(tbuf.at[d, pl.ds(j * k, k)],
                                acc.at[d].at[idx2.at[j]], add=True)
            return carry

        lax.fori_loop(0, nch, chunk, 0)
        plsc.subcore_barrier()

        @pl.when(sid == 0)
        def _():
            pltpu.sync_copy(acc, out_hbm.at[cid])

    return scatter


def _combine_tc(n):
    def body(p1_ref, p2_ref, p3_ref, p4_ref, c_ref, o_ref):
        p = (p1_ref[0] + p1_ref[1] + p2_ref[0] + p2_ref[1]
             + p3_ref[0] + p3_ref[1] + p4_ref[0] + p4_ref[1])
        o_ref[...] = c_ref[...] + p.T / jnp.float32(100.0)

    return pl.pallas_call(
        body,
        out_shape=jax.ShapeDtypeStruct((n, 3), jnp.float32),
    )


def kernel(h, coord, edge_index, coord_diff, edge_attr, W1, b1, W2, b2, W3):
    n, hnf = h.shape
    e = edge_index.shape[1]
    f32 = jnp.float32
    row = edge_index[0].astype(jnp.int32)
    col = edge_index[1].astype(jnp.int32)
    w1t = W1.T.astype(f32)
    w1a = w1t[:hnf]
    w1b = w1t[hnf:2 * hnf]
    w1c = w1t[2 * hnf:]
    ea_t = edge_attr.astype(f32).T
    cd_t = jnp.concatenate(
        [coord_diff.astype(f32).T, jnp.zeros((1, e), f32)], axis=0)
    b1r = b1.reshape(1, hnf).astype(f32)
    b2r = b2.reshape(1, hnf).astype(f32)
    w2t = W2.T.astype(f32)
    w3r = W3.reshape(1, hnf).astype(f32)
    zeros_3n = jnp.zeros((3, n), f32)

    A, B = _precompute_tc(n, hnf, 2000)(h.astype(f32), w1a, w1b)
    chunks = [38400, 121600, 121600, 38400]
    gats = {}
    mlps = {}
    scats = {}
    parts = []
    off = 0
    for ce in chunks:
        if ce not in gats:
            gats[ce] = _gather_add_sc(ce, hnf, 40)
            mlps[ce] = _mlp_tc(ce, hnf, 1280)
            scats[ce] = _scatter_sc(ce, n, 40)
        sl = slice(off, off + ce)
        s_i = gats[ce](A, B, row[sl], col[sl])
        t_i = mlps[ce](s_i, ea_t[:, sl], cd_t[:, sl], w1c, b1r, w2t, b2r, w3r)
        row2d_i = row[sl].reshape(ce // 40, 40)
        parts.append(scats[ce](t_i, row2d_i, zeros_3n))
        off += ce
    return _combine_tc(n)(*parts, coord.astype(f32))


# speedup vs baseline: 6.7768x; 1.0100x over previous
import functools

import jax
import jax.numpy as jnp
from jax import lax
from jax.experimental import pallas as pl
from jax.experimental.pallas import tpu as pltpu
from jax.experimental.pallas import tpu_sc as plsc

NC = 2
NS = 16
NW = NC * NS


def _precompute_tc(n, hnf, bn):
    def body(h_ref, wa_ref, wb_ref, a_ref, b_ref):
        hb = h_ref[...]
        a_ref[...] = jnp.dot(hb, wa_ref[...], preferred_element_type=jnp.float32)
        b_ref[...] = jnp.dot(hb, wb_ref[...], preferred_element_type=jnp.float32)

    return pl.pallas_call(
        body,
        grid=(n // bn,),
        in_specs=[
            pl.BlockSpec((bn, hnf), lambda i: (i, 0)),
            pl.BlockSpec((hnf, hnf), lambda i: (0, 0)),
            pl.BlockSpec((hnf, hnf), lambda i: (0, 0)),
        ],
        out_specs=[
            pl.BlockSpec((bn, hnf), lambda i: (i, 0)),
            pl.BlockSpec((bn, hnf), lambda i: (i, 0)),
        ],
        out_shape=[
            jax.ShapeDtypeStruct((n, hnf), jnp.float32),
            jax.ShapeDtypeStruct((n, hnf), jnp.float32),
        ],
    )


def _gather_add_sc(e, hnf, k):
    ew = e // NW
    nch = ew // k
    mesh = plsc.VectorSubcoreMesh(core_axis_name="c", subcore_axis_name="s")

    @functools.partial(
        pl.kernel,
        out_type=jax.ShapeDtypeStruct((e, hnf), jnp.float32),
        mesh=mesh,
        scratch_types=[
            pltpu.VMEM((ew,), jnp.int32),
            pltpu.VMEM((ew,), jnp.int32),
            pltpu.VMEM((k, hnf), jnp.float32),
            pltpu.VMEM((k, hnf), jnp.float32),
            pltpu.VMEM((k, hnf), jnp.float32),
            pltpu.VMEM((k, hnf), jnp.float32),
            pltpu.VMEM((k, hnf), jnp.float32),
            pltpu.VMEM((k, hnf), jnp.float32),
            pltpu.SemaphoreType.DMA,
            pltpu.SemaphoreType.DMA,
            pltpu.SemaphoreType.DMA,
            pltpu.SemaphoreType.DMA,
            pltpu.SemaphoreType.DMA,
            pltpu.SemaphoreType.DMA,
        ],
    )
    def gather_add(a_hbm, b_hbm, row_hbm, col_hbm, s_hbm,
                   idr, idc, ba0, ba1, bb0, bb1, sb0, sb1,
                   sma0, sma1, smb0, smb1, smw0, smw1):
        cid = lax.axis_index("c")
        sid = lax.axis_index("s")
        wid = sid * NC + cid
        base = wid * ew
        pltpu.sync_copy(row_hbm.at[pl.ds(base, ew)], idr)
        pltpu.sync_copy(col_hbm.at[pl.ds(base, ew)], idc)
        ba = (ba0, ba1)
        bb = (bb0, bb1)
        sb = (sb0, sb1)
        sma = (sma0, sma1)
        smb = (smb0, smb1)
        smw = (smw0, smw1)

        def issue(g, b):
            pltpu.make_async_copy(
                a_hbm.at[idr.at[pl.ds(g * k, k)]], ba[b], sma[b]).start()
            pltpu.make_async_copy(
                b_hbm.at[idc.at[pl.ds(g * k, k)]], bb[b], smb[b]).start()

        def addrows(b):
            def addrow(r, c2):
                for tt in range(hnf // 16):
                    sl = pl.ds(tt * 16, 16)
                    sb[b][r, sl] = ba[b][r, sl] + bb[b][r, sl]
                return c2

            lax.fori_loop(0, k, addrow, 0)

        def step(g, b):
            @pl.when(g + 1 < nch)
            def _():
                issue(g + 1, 1 - b)

            pltpu.make_async_copy(
                a_hbm.at[idr.at[pl.ds(g * k, k)]], ba[b], sma[b]).wait()
            pltpu.make_async_copy(
                b_hbm.at[idc.at[pl.ds(g * k, k)]], bb[b], smb[b]).wait()

            @pl.when(g >= 2)
            def _():
                pltpu.make_async_copy(
                    sb[b], s_hbm.at[pl.ds(base, k)], smw[b]).wait()

            addrows(b)
            pltpu.make_async_copy(
                sb[b], s_hbm.at[pl.ds(base + g * k, k)], smw[b]).start()

        issue(0, 0)

        def pair(t, carry):
            for b in range(2):
                step(t * 2 + b, b)
            return carry

        lax.fori_loop(0, nch // 2, pair, 0)
        if nch % 2:
            step(nch - 1, 0)
        pltpu.make_async_copy(sb0, s_hbm.at[pl.ds(base, k)], smw0).wait()
        pltpu.make_async_copy(sb1, s_hbm.at[pl.ds(base, k)], smw1).wait()

    return gather_add


def _mlp_tc(e, hnf, be, ob):
    def body(s_ref, ea_ref, cd_ref, w1c_ref, b1_ref, w2_ref, b2_ref, w3_ref,
             o_ref):
        eaw = lax.dot_general(ea_ref[...], w1c_ref[...],
                              (((0,), (0,)), ((), ())),
                              preferred_element_type=jnp.float32)
        u = s_ref[...] + eaw + b1_ref[...]
        u = u * lax.logistic(u)
        x = jnp.dot(u.astype(jnp.bfloat16), w2_ref[...].astype(jnp.bfloat16),
                    preferred_element_type=jnp.float32) + b2_ref[...]
        x = x * lax.logistic(x)
        m = lax.dot_general(w3_ref[...].astype(jnp.bfloat16),
                            x.astype(jnp.bfloat16), (((1,), (1,)), ((), ())),
                            preferred_element_type=jnp.float32)
        o_ref[...] = cd_ref[...] * m

    return pl.pallas_call(
        body,
        grid=(e // be,),
        in_specs=[
            pl.BlockSpec((be, hnf), lambda i: (i, 0)),
            pl.BlockSpec((3, be), lambda i: (0, ob + i)),
            pl.BlockSpec((4, be), lambda i: (0, ob + i)),
            pl.BlockSpec((3, hnf), lambda i: (0, 0)),
            pl.BlockSpec((1, hnf), lambda i: (0, 0)),
            pl.BlockSpec((hnf, hnf), lambda i: (0, 0)),
            pl.BlockSpec((1, hnf), lambda i: (0, 0)),
            pl.BlockSpec((1, hnf), lambda i: (0, 0)),
        ],
        out_specs=pl.BlockSpec((4, be), lambda i: (0, i)),
        out_shape=jax.ShapeDtypeStruct((4, e), jnp.float32),
    )


def _scatter_sc(ce, n, k):
    ew = ce // NW
    nch = ew // k
    mesh = plsc.VectorSubcoreMesh(core_axis_name="c", subcore_axis_name="s")

    @functools.partial(
        pl.kernel,
        out_type=jax.ShapeDtypeStruct((NC, 3, n), jnp.float32),
        mesh=mesh,
        compiler_params=pltpu.CompilerParams(use_tc_tiling_on_sc=False),
        scratch_types=[
            pltpu.VMEM((nch, k), jnp.int32),
            pltpu.VMEM((3, ew), jnp.float32),
            pltpu.VMEM_SHARED((3, n), jnp.float32),
        ],
    )
    def scatter(trans_hbm, row2d_hbm, zero_hbm, out_hbm, idx2, tbuf, acc):
        cid = lax.axis_index("c")
        sid = lax.axis_index("s")
        wid = sid * NC + cid

        @pl.when(sid == 0)
        def _():
            pltpu.sync_copy(zero_hbm, acc)

        pltpu.sync_copy(row2d_hbm.at[pl.ds(wid * nch, nch)], idx2)
        for d in range(3):
            pltpu.sync_copy(trans_hbm.at[d, pl.ds(wid * ew, ew)], tbuf.at[d])
        plsc.subcore_barrier()

        def chunk(j, carry):
            for d in range(3):
                pltpu.sync_copy(tbuf.at[d, pl.ds(j * k, k)],
                                acc.at[d].at[idx2.at[j]], add=True)
            return carry

        lax.fori_loop(0, nch, chunk, 0)
        plsc.subcore_barrier()

        @pl.when(sid == 0)
        def _():
            pltpu.sync_copy(acc, out_hbm.at[cid])

    return scatter


def _combine_tc(n):
    def body(p1_ref, p2_ref, p3_ref, p4_ref, c_ref, o_ref):
        p = (p1_ref[0] + p1_ref[1] + p2_ref[0] + p2_ref[1]
             + p3_ref[0] + p3_ref[1] + p4_ref[0] + p4_ref[1])
        o_ref[...] = c_ref[...] + p.T / jnp.float32(100.0)

    return pl.pallas_call(
        body,
        out_shape=jax.ShapeDtypeStruct((n, 3), jnp.float32),
    )


def kernel(h, coord, edge_index, coord_diff, edge_attr, W1, b1, W2, b2, W3):
    n, hnf = h.shape
    e = edge_index.shape[1]
    f32 = jnp.float32
    row = edge_index[0].astype(jnp.int32)
    col = edge_index[1].astype(jnp.int32)
    w1t = W1.T.astype(f32)
    w1a = w1t[:hnf]
    w1b = w1t[hnf:2 * hnf]
    w1c = w1t[2 * hnf:]
    ea_t = edge_attr.astype(f32).T
    cd_t = jnp.concatenate(
        [coord_diff.astype(f32).T, jnp.zeros((1, e), f32)], axis=0)
    b1r = b1.reshape(1, hnf).astype(f32)
    b2r = b2.reshape(1, hnf).astype(f32)
    w2t = W2.T.astype(f32)
    w3r = W3.reshape(1, hnf).astype(f32)
    zeros_3n = jnp.zeros((3, n), f32)

    A, B = _precompute_tc(n, hnf, 2000)(h.astype(f32), w1a, w1b)
    chunks = [38400, 121600, 121600, 38400]
    gats = {}
    scats = {}
    parts = []
    off = 0
    for ce in chunks:
        if ce not in gats:
            gats[ce] = _gather_add_sc(ce, hnf, 40)
            scats[ce] = _scatter_sc(ce, n, 40)
        sl = slice(off, off + ce)
        s_i = gats[ce](A, B, row[sl], col[sl])
        t_i = _mlp_tc(ce, hnf, 1280, off // 1280)(
            s_i, ea_t, cd_t, w1c, b1r, w2t, b2r, w3r)
        row2d_i = row[sl].reshape(ce // 40, 40)
        parts.append(scats[ce](t_i, row2d_i, zeros_3n))
        off += ce
    return _combine_tc(n)(*parts, coord.astype(f32))


# speedup vs baseline: 6.7892x; 1.0018x over previous
import functools

import jax
import jax.numpy as jnp
from jax import lax
from jax.experimental import pallas as pl
from jax.experimental.pallas import tpu as pltpu
from jax.experimental.pallas import tpu_sc as plsc

NC = 2
NS = 16
NW = NC * NS


def _precompute_tc(n, hnf, bn):
    def body(h_ref, wa_ref, wb_ref, a_ref, b_ref):
        hb = h_ref[...]
        a_ref[...] = jnp.dot(hb, wa_ref[...], preferred_element_type=jnp.float32)
        b_ref[...] = jnp.dot(hb, wb_ref[...], preferred_element_type=jnp.float32)

    return pl.pallas_call(
        body,
        grid=(n // bn,),
        in_specs=[
            pl.BlockSpec((bn, hnf), lambda i: (i, 0)),
            pl.BlockSpec((hnf, hnf), lambda i: (0, 0)),
            pl.BlockSpec((hnf, hnf), lambda i: (0, 0)),
        ],
        out_specs=[
            pl.BlockSpec((bn, hnf), lambda i: (i, 0)),
            pl.BlockSpec((bn, hnf), lambda i: (i, 0)),
        ],
        out_shape=[
            jax.ShapeDtypeStruct((n, hnf), jnp.float32),
            jax.ShapeDtypeStruct((n, hnf), jnp.float32),
        ],
    )


def _gather_add_sc(e, hnf, k):
    ew = e // NW
    nch = ew // k
    mesh = plsc.VectorSubcoreMesh(core_axis_name="c", subcore_axis_name="s")

    @functools.partial(
        pl.kernel,
        out_type=jax.ShapeDtypeStruct((e, hnf), jnp.float32),
        mesh=mesh,
        scratch_types=[
            pltpu.VMEM((ew,), jnp.int32),
            pltpu.VMEM((ew,), jnp.int32),
            pltpu.VMEM((k, hnf), jnp.float32),
            pltpu.VMEM((k, hnf), jnp.float32),
            pltpu.VMEM((k, hnf), jnp.float32),
            pltpu.VMEM((k, hnf), jnp.float32),
            pltpu.VMEM((k, hnf), jnp.float32),
            pltpu.VMEM((k, hnf), jnp.float32),
            pltpu.SemaphoreType.DMA,
            pltpu.SemaphoreType.DMA,
            pltpu.SemaphoreType.DMA,
            pltpu.SemaphoreType.DMA,
            pltpu.SemaphoreType.DMA,
            pltpu.SemaphoreType.DMA,
        ],
    )
    def gather_add(a_hbm, b_hbm, row_hbm, col_hbm, s_hbm,
                   idr, idc, ba0, ba1, bb0, bb1, sb0, sb1,
                   sma0, sma1, smb0, smb1, smw0, smw1):
        cid = lax.axis_index("c")
        sid = lax.axis_index("s")
        wid = sid * NC + cid
        base = wid * ew
        pltpu.sync_copy(row_hbm.at[pl.ds(base, ew)], idr)
        pltpu.sync_copy(col_hbm.at[pl.ds(base, ew)], idc)
        ba = (ba0, ba1)
        bb = (bb0, bb1)
        sb = (sb0, sb1)
        sma = (sma0, sma1)
        smb = (smb0, smb1)
        smw = (smw0, smw1)

        def issue(g, b):
            pltpu.make_async_copy(
                a_hbm.at[idr.at[pl.ds(g * k, k)]], ba[b], sma[b]).start()
            pltpu.make_async_copy(
                b_hbm.at[idc.at[pl.ds(g * k, k)]], bb[b], smb[b]).start()

        def addrows(b):
            def addrow(r, c2):
                for tt in range(hnf // 16):
                    sl = pl.ds(tt * 16, 16)
                    sb[b][r, sl] = ba[b][r, sl] + bb[b][r, sl]
                return c2

            lax.fori_loop(0, k, addrow, 0)

        def step(g, b):
            @pl.when(g + 1 < nch)
            def _():
                issue(g + 1, 1 - b)

            pltpu.make_async_copy(
                a_hbm.at[idr.at[pl.ds(g * k, k)]], ba[b], sma[b]).wait()
            pltpu.make_async_copy(
                b_hbm.at[idc.at[pl.ds(g * k, k)]], bb[b], smb[b]).wait()

            @pl.when(g >= 2)
            def _():
                pltpu.make_async_copy(
                    sb[b], s_hbm.at[pl.ds(base, k)], smw[b]).wait()

            addrows(b)
            pltpu.make_async_copy(
                sb[b], s_hbm.at[pl.ds(base + g * k, k)], smw[b]).start()

        issue(0, 0)

        def pair(t, carry):
            for b in range(2):
                step(t * 2 + b, b)
            return carry

        lax.fori_loop(0, nch // 2, pair, 0)
        if nch % 2:
            step(nch - 1, 0)
        pltpu.make_async_copy(sb0, s_hbm.at[pl.ds(base, k)], smw0).wait()
        pltpu.make_async_copy(sb1, s_hbm.at[pl.ds(base, k)], smw1).wait()

    return gather_add


def _mlp_tc(e, hnf, be, ob):
    def body(s_ref, ea_ref, cd_ref, w1c_ref, b1_ref, w2_ref, b2_ref, w3_ref,
             o_ref):
        eaw = lax.dot_general(ea_ref[...], w1c_ref[...],
                              (((0,), (0,)), ((), ())),
                              preferred_element_type=jnp.float32)
        u = s_ref[...] + eaw + b1_ref[...]
        u = u * (jnp.float32(0.5) + jnp.float32(0.5)
                 * lax.tanh(u * jnp.float32(0.5)))
        x = jnp.dot(u.astype(jnp.bfloat16), w2_ref[...].astype(jnp.bfloat16),
                    preferred_element_type=jnp.float32) + b2_ref[...]
        x = x * (jnp.float32(0.5) + jnp.float32(0.5)
                 * lax.tanh(x * jnp.float32(0.5)))
        m = lax.dot_general(w3_ref[...].astype(jnp.bfloat16),
                            x.astype(jnp.bfloat16), (((1,), (1,)), ((), ())),
                            preferred_element_type=jnp.float32)
        o_ref[...] = cd_ref[...] * m

    return pl.pallas_call(
        body,
        grid=(e // be,),
        in_specs=[
            pl.BlockSpec((be, hnf), lambda i: (i, 0)),
            pl.BlockSpec((3, be), lambda i: (0, ob + i)),
            pl.BlockSpec((4, be), lambda i: (0, ob + i)),
            pl.BlockSpec((3, hnf), lambda i: (0, 0)),
            pl.BlockSpec((1, hnf), lambda i: (0, 0)),
            pl.BlockSpec((hnf, hnf), lambda i: (0, 0)),
            pl.BlockSpec((1, hnf), lambda i: (0, 0)),
            pl.BlockSpec((1, hnf), lambda i: (0, 0)),
        ],
        out_specs=pl.BlockSpec((4, be), lambda i: (0, i)),
        out_shape=jax.ShapeDtypeStruct((4, e), jnp.float32),
    )


def _scatter_sc(ce, n, k):
    ew = ce // NW
    nch = ew // k
    mesh = plsc.VectorSubcoreMesh(core_axis_name="c", subcore_axis_name="s")

    @functools.partial(
        pl.kernel,
        out_type=jax.ShapeDtypeStruct((NC, 3, n), jnp.float32),
        mesh=mesh,
        compiler_params=pltpu.CompilerParams(use_tc_tiling_on_sc=False),
        scratch_types=[
            pltpu.VMEM((nch, k), jnp.int32),
            pltpu.VMEM((3, ew), jnp.float32),
            pltpu.VMEM_SHARED((3, n), jnp.float32),
        ],
    )
    def scatter(trans_hbm, row2d_hbm, zero_hbm, out_hbm, idx2, tbuf, acc):
        cid = lax.axis_index("c")
        sid = lax.axis_index("s")
        wid = sid * NC + cid

        @pl.when(sid == 0)
        def _():
            pltpu.sync_copy(zero_hbm, acc)

        pltpu.sync_copy(row2d_hbm.at[pl.ds(wid * nch, nch)], idx2)
        for d in range(3):
            pltpu.sync_copy(trans_hbm.at[d, pl.ds(wid * ew, ew)], tbuf.at[d])
        plsc.subcore_barrier()

        def chunk(j, carry):
            for d in range(3):
                pltpu.sync_copy(tbuf.at[d, pl.ds(j * k, k)],
                                acc.at[d].at[idx2.at[j]], add=True)
            return carry

        lax.fori_loop(0, nch, chunk, 0)
        plsc.subcore_barrier()

        @pl.when(sid == 0)
        def _():
            pltpu.sync_copy(acc, out_hbm.at[cid])

    return scatter


def _combine_tc(n):
    def body(p1_ref, p2_ref, p3_ref, p4_ref, c_ref, o_ref):
        p = (p1_ref[0] + p1_ref[1] + p2_ref[0] + p2_ref[1]
             + p3_ref[0] + p3_ref[1] + p4_ref[0] + p4_ref[1])
        o_ref[...] = c_ref[...] + p.T / jnp.float32(100.0)

    return pl.pallas_call(
        body,
        out_shape=jax.ShapeDtypeStruct((n, 3), jnp.float32),
    )


def kernel(h, coord, edge_index, coord_diff, edge_attr, W1, b1, W2, b2, W3):
    n, hnf = h.shape
    e = edge_index.shape[1]
    f32 = jnp.float32
    row = edge_index[0].astype(jnp.int32)
    col = edge_index[1].astype(jnp.int32)
    w1t = W1.T.astype(f32)
    w1a = w1t[:hnf]
    w1b = w1t[hnf:2 * hnf]
    w1c = w1t[2 * hnf:]
    ea_t = edge_attr.astype(f32).T
    cd_t = jnp.concatenate(
        [coord_diff.astype(f32).T, jnp.zeros((1, e), f32)], axis=0)
    b1r = b1.reshape(1, hnf).astype(f32)
    b2r = b2.reshape(1, hnf).astype(f32)
    w2t = W2.T.astype(f32)
    w3r = W3.reshape(1, hnf).astype(f32)
    zeros_3n = jnp.zeros((3, n), f32)

    A, B = _precompute_tc(n, hnf, 2000)(h.astype(f32), w1a, w1b)
    chunks = [38400, 121600, 121600, 38400]
    gats = {}
    scats = {}
    parts = []
    off = 0
    for ce in chunks:
        if ce not in gats:
            gats[ce] = _gather_add_sc(ce, hnf, 40)
            scats[ce] = _scatter_sc(ce, n, 40)
        sl = slice(off, off + ce)
        s_i = gats[ce](A, B, row[sl], col[sl])
        t_i = _mlp_tc(ce, hnf, 1280, off // 1280)(
            s_i, ea_t, cd_t, w1c, b1r, w2t, b2r, w3r)
        row2d_i = row[sl].reshape(ce // 40, 40)
        parts.append(scats[ce](t_i, row2d_i, zeros_3n))
        off += ce
    return _combine_tc(n)(*parts, coord.astype(f32))


# speedup vs baseline: 7.0390x; 1.0368x over previous
import functools

import jax
import jax.numpy as jnp
from jax import lax
from jax.experimental import pallas as pl
from jax.experimental.pallas import tpu as pltpu
from jax.experimental.pallas import tpu_sc as plsc

NC = 2
NS = 16
NW = NC * NS


def _precompute_tc(n, hnf, bn):
    def body(h_ref, wa_ref, wb_ref, a_ref, b_ref):
        hb = h_ref[...]
        a_ref[...] = jnp.dot(hb, wa_ref[...], preferred_element_type=jnp.float32)
        b_ref[...] = jnp.dot(hb, wb_ref[...], preferred_element_type=jnp.float32)

    return pl.pallas_call(
        body,
        grid=(n // bn,),
        in_specs=[
            pl.BlockSpec((bn, hnf), lambda i: (i, 0)),
            pl.BlockSpec((hnf, hnf), lambda i: (0, 0)),
            pl.BlockSpec((hnf, hnf), lambda i: (0, 0)),
        ],
        out_specs=[
            pl.BlockSpec((bn, hnf), lambda i: (i, 0)),
            pl.BlockSpec((bn, hnf), lambda i: (i, 0)),
        ],
        out_shape=[
            jax.ShapeDtypeStruct((n, hnf), jnp.float32),
            jax.ShapeDtypeStruct((n, hnf), jnp.float32),
        ],
    )


def _gather_add_sc(ce, hnf, k, goff):
    ew = ce // NW
    nch = ew // k
    mesh = plsc.VectorSubcoreMesh(core_axis_name="c", subcore_axis_name="s")

    @functools.partial(
        pl.kernel,
        out_type=jax.ShapeDtypeStruct((ce, hnf), jnp.float32),
        mesh=mesh,
        scratch_types=[
            pltpu.VMEM((ew,), jnp.int32),
            pltpu.VMEM((ew,), jnp.int32),
            pltpu.VMEM((k, hnf), jnp.float32),
            pltpu.VMEM((k, hnf), jnp.float32),
            pltpu.VMEM((k, hnf), jnp.float32),
            pltpu.VMEM((k, hnf), jnp.float32),
            pltpu.VMEM((k, hnf), jnp.float32),
            pltpu.VMEM((k, hnf), jnp.float32),
            pltpu.SemaphoreType.DMA,
            pltpu.SemaphoreType.DMA,
            pltpu.SemaphoreType.DMA,
            pltpu.SemaphoreType.DMA,
            pltpu.SemaphoreType.DMA,
            pltpu.SemaphoreType.DMA,
        ],
    )
    def gather_add(a_hbm, b_hbm, row_hbm, col_hbm, s_hbm,
                   idr, idc, ba0, ba1, bb0, bb1, sb0, sb1,
                   sma0, sma1, smb0, smb1, smw0, smw1):
        cid = lax.axis_index("c")
        sid = lax.axis_index("s")
        wid = sid * NC + cid
        base = wid * ew
        pltpu.sync_copy(row_hbm.at[pl.ds(goff + base, ew)], idr)
        pltpu.sync_copy(col_hbm.at[pl.ds(goff + base, ew)], idc)
        ba = (ba0, ba1)
        bb = (bb0, bb1)
        sb = (sb0, sb1)
        sma = (sma0, sma1)
        smb = (smb0, smb1)
        smw = (smw0, smw1)

        def issue(g, b):
            pltpu.make_async_copy(
                a_hbm.at[idr.at[pl.ds(g * k, k)]], ba[b], sma[b]).start()
            pltpu.make_async_copy(
                b_hbm.at[idc.at[pl.ds(g * k, k)]], bb[b], smb[b]).start()

        def addrows(b):
            def addrow(r, c2):
                for tt in range(hnf // 16):
                    sl = pl.ds(tt * 16, 16)
                    sb[b][r, sl] = ba[b][r, sl] + bb[b][r, sl]
                return c2

            lax.fori_loop(0, k, addrow, 0)

        def step(g, b):
            @pl.when(g + 1 < nch)
            def _():
                issue(g + 1, 1 - b)

            pltpu.make_async_copy(
                a_hbm.at[idr.at[pl.ds(g * k, k)]], ba[b], sma[b]).wait()
            pltpu.make_async_copy(
                b_hbm.at[idc.at[pl.ds(g * k, k)]], bb[b], smb[b]).wait()

            @pl.when(g >= 2)
            def _():
                pltpu.make_async_copy(
                    sb[b], s_hbm.at[pl.ds(base, k)], smw[b]).wait()

            addrows(b)
            pltpu.make_async_copy(
                sb[b], s_hbm.at[pl.ds(base + g * k, k)], smw[b]).start()

        issue(0, 0)

        def pair(t, carry):
            for b in range(2):
                step(t * 2 + b, b)
            return carry

        lax.fori_loop(0, nch // 2, pair, 0)
        if nch % 2:
            step(nch - 1, 0)
        pltpu.make_async_copy(sb0, s_hbm.at[pl.ds(base, k)], smw0).wait()
        pltpu.make_async_copy(sb1, s_hbm.at[pl.ds(base, k)], smw1).wait()

    return gather_add


def _mlp_tc(e, hnf, be, ob):
    def body(s_ref, ea_ref, cd_ref, w1c_ref, b1_ref, w2_ref, b2_ref, w3_ref,
             o_ref):
        eaw = lax.dot_general(ea_ref[...], w1c_ref[...],
                              (((0,), (0,)), ((), ())),
                              preferred_element_type=jnp.float32)
        u = s_ref[...] + eaw + b1_ref[...]
        u = u * (jnp.float32(0.5) + jnp.float32(0.5)
                 * lax.tanh(u * jnp.float32(0.5)))
        x = jnp.dot(u.astype(jnp.bfloat16), w2_ref[...].astype(jnp.bfloat16),
                    preferred_element_type=jnp.float32) + b2_ref[...]
        x = x * (jnp.float32(0.5) + jnp.float32(0.5)
                 * lax.tanh(x * jnp.float32(0.5)))
        m = lax.dot_general(w3_ref[...].astype(jnp.bfloat16),
                            x.astype(jnp.bfloat16), (((1,), (1,)), ((), ())),
                            preferred_element_type=jnp.float32)
        o_ref[...] = cd_ref[...] * m

    return pl.pallas_call(
        body,
        grid=(e // be,),
        in_specs=[
            pl.BlockSpec((be, hnf), lambda i: (i, 0)),
            pl.BlockSpec((3, be), lambda i: (0, ob + i)),
            pl.BlockSpec((4, be), lambda i: (0, ob + i)),
            pl.BlockSpec((3, hnf), lambda i: (0, 0)),
            pl.BlockSpec((1, hnf), lambda i: (0, 0)),
            pl.BlockSpec((hnf, hnf), lambda i: (0, 0)),
            pl.BlockSpec((1, hnf), lambda i: (0, 0)),
            pl.BlockSpec((1, hnf), lambda i: (0, 0)),
        ],
        out_specs=pl.BlockSpec((4, be), lambda i: (0, i)),
        out_shape=jax.ShapeDtypeStruct((4, e), jnp.float32),
    )


def _scatter_sc(ce, n, k, coff):
    ew = ce // NW
    nch = ew // k
    choff = coff // k
    mesh = plsc.VectorSubcoreMesh(core_axis_name="c", subcore_axis_name="s")

    @functools.partial(
        pl.kernel,
        out_type=jax.ShapeDtypeStruct((NC, 3, n), jnp.float32),
        mesh=mesh,
        compiler_params=pltpu.CompilerParams(use_tc_tiling_on_sc=False),
        scratch_types=[
            pltpu.VMEM((nch, k), jnp.int32),
            pltpu.VMEM((3, ew), jnp.float32),
            pltpu.VMEM_SHARED((3, n), jnp.float32),
        ],
    )
    def scatter(trans_hbm, row2d_hbm, zero_hbm, out_hbm, idx2, tbuf, acc):
        cid = lax.axis_index("c")
        sid = lax.axis_index("s")
        wid = sid * NC + cid

        @pl.when(sid == 0)
        def _():
            pltpu.sync_copy(zero_hbm, acc)

        pltpu.sync_copy(row2d_hbm.at[pl.ds(choff + wid * nch, nch)], idx2)
        for d in range(3):
            pltpu.sync_copy(trans_hbm.at[d, pl.ds(wid * ew, ew)], tbuf.at[d])
        plsc.subcore_barrier()

        def chunk(j, carry):
            for d in range(3):
                pltpu.sync_copy(tbuf.at[d, pl.ds(j * k, k)],
                                acc.at[d].at[idx2.at[j]], add=True)
            return carry

        lax.fori_loop(0, nch, chunk, 0)
        plsc.subcore_barrier()

        @pl.when(sid == 0)
        def _():
            pltpu.sync_copy(acc, out_hbm.at[cid])

    return scatter


def _combine_tc(n):
    def body(p1_ref, p2_ref, p3_ref, p4_ref, c_ref, o_ref):
        p = (p1_ref[0] + p1_ref[1] + p2_ref[0] + p2_ref[1]
             + p3_ref[0] + p3_ref[1] + p4_ref[0] + p4_ref[1])
        o_ref[...] = c_ref[...] + p.T / jnp.float32(100.0)

    return pl.pallas_call(
        body,
        out_shape=jax.ShapeDtypeStruct((n, 3), jnp.float32),
    )


def kernel(h, coord, edge_index, coord_diff, edge_attr, W1, b1, W2, b2, W3):
    n, hnf = h.shape
    e = edge_index.shape[1]
    f32 = jnp.float32
    row = edge_index[0].astype(jnp.int32)
    col = edge_index[1].astype(jnp.int32)
    w1t = W1.T.astype(f32)
    w1a = w1t[:hnf]
    w1b = w1t[hnf:2 * hnf]
    w1c = w1t[2 * hnf:]
    ea_t = edge_attr.astype(f32).T
    cd_t = jnp.concatenate(
        [coord_diff.astype(f32).T, jnp.zeros((1, e), f32)], axis=0)
    b1r = b1.reshape(1, hnf).astype(f32)
    b2r = b2.reshape(1, hnf).astype(f32)
    w2t = W2.T.astype(f32)
    w3r = W3.reshape(1, hnf).astype(f32)
    zeros_3n = jnp.zeros((3, n), f32)

    A, B = _precompute_tc(n, hnf, 2000)(h.astype(f32), w1a, w1b)
    chunks = [38400, 121600, 121600, 38400]
    row2d = row.reshape(e // 40, 40)
    parts = []
    off = 0
    for ce in chunks:
        s_i = _gather_add_sc(ce, hnf, 40, off)(A, B, row, col)
        t_i = _mlp_tc(ce, hnf, 1280, off // 1280)(
            s_i, ea_t, cd_t, w1c, b1r, w2t, b2r, w3r)
        parts.append(_scatter_sc(ce, n, 40, off)(t_i, row2d, zeros_3n))
        off += ce
    return _combine_tc(n)(*parts, coord.astype(f32))


# speedup vs baseline: 8.1488x; 1.1577x over previous
import functools

import jax
import jax.numpy as jnp
from jax import lax
from jax.experimental import pallas as pl
from jax.experimental.pallas import tpu as pltpu
from jax.experimental.pallas import tpu_sc as plsc

NC = 2
NS = 16
NW = NC * NS


def _precompute_tc(n, hnf, bn):
    def body(h_ref, wa_ref, wb_ref, a_ref, b_ref):
        hb = h_ref[...]
        a_ref[...] = jnp.dot(hb, wa_ref[...], preferred_element_type=jnp.float32)
        b_ref[...] = jnp.dot(hb, wb_ref[...], preferred_element_type=jnp.float32)

    return pl.pallas_call(
        body,
        grid=(n // bn,),
        in_specs=[
            pl.BlockSpec((bn, hnf), lambda i: (i, 0)),
            pl.BlockSpec((hnf, hnf), lambda i: (0, 0)),
            pl.BlockSpec((hnf, hnf), lambda i: (0, 0)),
        ],
        out_specs=[
            pl.BlockSpec((bn, hnf), lambda i: (i, 0)),
            pl.BlockSpec((bn, hnf), lambda i: (i, 0)),
        ],
        out_shape=[
            jax.ShapeDtypeStruct((n, hnf), jnp.float32),
            jax.ShapeDtypeStruct((n, hnf), jnp.float32),
        ],
    )


def _gather_add_sc(ce, hnf, k, goff):
    ew = ce // NW
    nch = ew // k
    mesh = plsc.VectorSubcoreMesh(core_axis_name="c", subcore_axis_name="s")

    @functools.partial(
        pl.kernel,
        out_type=jax.ShapeDtypeStruct((ce, hnf), jnp.float32),
        mesh=mesh,
        scratch_types=[
            pltpu.VMEM((ew,), jnp.int32),
            pltpu.VMEM((ew,), jnp.int32),
            pltpu.VMEM((k, hnf), jnp.float32),
            pltpu.VMEM((k, hnf), jnp.float32),
            pltpu.VMEM((k, hnf), jnp.float32),
            pltpu.VMEM((k, hnf), jnp.float32),
            pltpu.VMEM((k, hnf), jnp.float32),
            pltpu.VMEM((k, hnf), jnp.float32),
            pltpu.SemaphoreType.DMA,
            pltpu.SemaphoreType.DMA,
            pltpu.SemaphoreType.DMA,
            pltpu.SemaphoreType.DMA,
            pltpu.SemaphoreType.DMA,
            pltpu.SemaphoreType.DMA,
        ],
    )
    def gather_add(a_hbm, b_hbm, row_hbm, col_hbm, s_hbm,
                   idr, idc, ba0, ba1, bb0, bb1, sb0, sb1,
                   sma0, sma1, smb0, smb1, smw0, smw1):
        cid = lax.axis_index("c")
        sid = lax.axis_index("s")
        wid = sid * NC + cid
        base = wid * ew
        pltpu.sync_copy(row_hbm.at[pl.ds(goff + base, ew)], idr)
        pltpu.sync_copy(col_hbm.at[pl.ds(goff + base, ew)], idc)
        ba = (ba0, ba1)
        bb = (bb0, bb1)
        sb = (sb0, sb1)
        sma = (sma0, sma1)
        smb = (smb0, smb1)
        smw = (smw0, smw1)

        def issue(g, b):
            pltpu.make_async_copy(
                a_hbm.at[idr.at[pl.ds(g * k, k)]], ba[b], sma[b]).start()
            pltpu.make_async_copy(
                b_hbm.at[idc.at[pl.ds(g * k, k)]], bb[b], smb[b]).start()

        def addrows(b):
            def addrow(r, c2):
                for tt in range(hnf // 16):
                    sl = pl.ds(tt * 16, 16)
                    sb[b][r, sl] = ba[b][r, sl] + bb[b][r, sl]
                return c2

            lax.fori_loop(0, k, addrow, 0)

        def step(g, b):
            @pl.when(g + 1 < nch)
            def _():
                issue(g + 1, 1 - b)

            pltpu.make_async_copy(
                a_hbm.at[idr.at[pl.ds(g * k, k)]], ba[b], sma[b]).wait()
            pltpu.make_async_copy(
                b_hbm.at[idc.at[pl.ds(g * k, k)]], bb[b], smb[b]).wait()

            @pl.when(g >= 2)
            def _():
                pltpu.make_async_copy(
                    sb[b], s_hbm.at[pl.ds(base, k)], smw[b]).wait()

            addrows(b)
            pltpu.make_async_copy(
                sb[b], s_hbm.at[pl.ds(base + g * k, k)], smw[b]).start()

        issue(0, 0)

        def pair(t, carry):
            for b in range(2):
                step(t * 2 + b, b)
            return carry

        lax.fori_loop(0, nch // 2, pair, 0)
        if nch % 2:
            step(nch - 1, 0)
        pltpu.make_async_copy(sb0, s_hbm.at[pl.ds(base, k)], smw0).wait()
        pltpu.make_async_copy(sb1, s_hbm.at[pl.ds(base, k)], smw1).wait()

    return gather_add


def _mlp_tc(e, hnf, be, ob):
    def body(s_ref, ea_ref, cd_ref, w1c_ref, b1_ref, w2_ref, b2_ref, w3_ref,
             o_ref):
        eaw = lax.dot_general(ea_ref[...], w1c_ref[...],
                              (((0,), (0,)), ((), ())),
                              preferred_element_type=jnp.float32)
        u = s_ref[...] + eaw + b1_ref[...]
        u = u * (jnp.float32(0.5) + jnp.float32(0.5)
                 * lax.tanh(u * jnp.float32(0.5)))
        x = jnp.dot(u.astype(jnp.bfloat16), w2_ref[...].astype(jnp.bfloat16),
                    preferred_element_type=jnp.float32) + b2_ref[...]
        x = x * (jnp.float32(0.5) + jnp.float32(0.5)
                 * lax.tanh(x * jnp.float32(0.5)))
        m = lax.dot_general(w3_ref[...].astype(jnp.bfloat16),
                            x.astype(jnp.bfloat16), (((1,), (1,)), ((), ())),
                            preferred_element_type=jnp.float32)
        o_ref[...] = cd_ref[...] * m

    return pl.pallas_call(
        body,
        grid=(e // be,),
        in_specs=[
            pl.BlockSpec((be, hnf), lambda i: (i, 0)),
            pl.BlockSpec((3, be), lambda i: (0, ob + i)),
            pl.BlockSpec((4, be), lambda i: (0, ob + i)),
            pl.BlockSpec((3, hnf), lambda i: (0, 0)),
            pl.BlockSpec((1, hnf), lambda i: (0, 0)),
            pl.BlockSpec((hnf, hnf), lambda i: (0, 0)),
            pl.BlockSpec((1, hnf), lambda i: (0, 0)),
            pl.BlockSpec((1, hnf), lambda i: (0, 0)),
        ],
        out_specs=pl.BlockSpec((4, be), lambda i: (0, i)),
        out_shape=jax.ShapeDtypeStruct((4, e), jnp.float32),
    )


def _scatter_sc(ce, n, k, coff):
    ew = ce // NW
    nch = ew // k
    choff = coff // k
    mesh = plsc.VectorSubcoreMesh(core_axis_name="c", subcore_axis_name="s")

    @functools.partial(
        pl.kernel,
        out_type=jax.ShapeDtypeStruct((NC, 3, n), jnp.float32),
        mesh=mesh,
        compiler_params=pltpu.CompilerParams(use_tc_tiling_on_sc=False),
        scratch_types=[
            pltpu.VMEM((nch, k), jnp.int32),
            pltpu.VMEM((3, ew), jnp.float32),
            pltpu.VMEM_SHARED((3, n), jnp.float32),
        ],
    )
    def scatter(trans_hbm, row2d_hbm, zero_hbm, out_hbm, idx2, tbuf, acc):
        cid = lax.axis_index("c")
        sid = lax.axis_index("s")
        wid = sid * NC + cid

        @pl.when(sid == 0)
        def _():
            pltpu.sync_copy(zero_hbm, acc)

        pltpu.sync_copy(row2d_hbm.at[pl.ds(choff + wid * nch, nch)], idx2)
        for d in range(3):
            pltpu.sync_copy(trans_hbm.at[d, pl.ds(wid * ew, ew)], tbuf.at[d])
        plsc.subcore_barrier()

        def chunk(j, carry):
            for d in range(3):
                pltpu.sync_copy(tbuf.at[d, pl.ds(j * k, k)],
                                acc.at[d].at[idx2.at[j]], add=True)
            return carry

        lax.fori_loop(0, nch, chunk, 0)
        plsc.subcore_barrier()

        @pl.when(sid == 0)
        def _():
            pltpu.sync_copy(acc, out_hbm.at[cid])

    return scatter


def _combine_tc(n):
    def body(p1_ref, p2_ref, p3_ref, p4_ref, c_ref, o_ref):
        p = (p1_ref[0] + p1_ref[1] + p2_ref[0] + p2_ref[1]
             + p3_ref[0] + p3_ref[1] + p4_ref[0] + p4_ref[1])
        o_ref[...] = c_ref[...] + p.T / jnp.float32(100.0)

    return pl.pallas_call(
        body,
        out_shape=jax.ShapeDtypeStruct((n, 3), jnp.float32),
    )


def kernel(h, coord, edge_index, coord_diff, edge_attr, W1, b1, W2, b2, W3):
    n, hnf = h.shape
    e = edge_index.shape[1]
    f32 = jnp.float32
    row = edge_index[0].astype(jnp.int32)
    col = edge_index[1].astype(jnp.int32)
    w1t = W1.T.astype(f32)
    w1a = w1t[:hnf]
    w1b = w1t[hnf:2 * hnf]
    w1c = w1t[2 * hnf:]
    ea_t = edge_attr.astype(f32).T
    cd_t = jnp.concatenate(
        [coord_diff.astype(f32).T, jnp.zeros((1, e), f32)], axis=0)
    b1r = b1.reshape(1, hnf).astype(f32)
    b2r = b2.reshape(1, hnf).astype(f32)
    w2t = W2.T.astype(f32)
    w3r = W3.reshape(1, hnf).astype(f32)
    zeros_3n = jnp.zeros((3, n), f32)

    A, B = _precompute_tc(n, hnf, 2000)(h.astype(f32), w1a, w1b)
    chunks = [40960, 117760, 120320, 40960]
    row2d = row.reshape(e // 40, 40)
    parts = []
    off = 0
    for ce in chunks:
        s_i = _gather_add_sc(ce, hnf, 80, off)(A, B, row, col)
        t_i = _mlp_tc(ce, hnf, 2560, off // 2560)(
            s_i, ea_t, cd_t, w1c, b1r, w2t, b2r, w3r)
        parts.append(_scatter_sc(ce, n, 40, off)(t_i, row2d, zeros_3n))
        off += ce
    return _combine_tc(n)(*parts, coord.astype(f32))
